# baseline (speedup 1.0000x reference)
import functools
import math

import jax
import jax.numpy as jnp
from jax import lax
from jax.experimental import pallas as pl
from jax.experimental.pallas import tpu as pltpu
from jax.experimental.pallas import tpu_sc as plsc

N = 10000
E = 320000
D = 128
K1 = int(math.ceil(N * 0.8))
K2 = int(math.ceil(K1 * 0.7))

NC, NS, L = 2, 16, 16
NW = NC * NS

ECH = 128
NCHUNK_RAW = -(-E // ECH)
CPT = -(-NCHUNK_RAW // NW)
NCHUNK = CPT * NW
EP = NCHUNK * ECH
IBLK = 16

NPAD1 = 10112
NPAD2 = 8064
NPAD3 = 5632

_CP = pltpu.CompilerParams(needs_layout_passes=False)

_mesh = lambda: plsc.VectorSubcoreMesh(core_axis_name="c", subcore_axis_name="s")

_IOTA = lambda: lax.iota(jnp.int32, L)


def _mo8(v):
    return pl.multiple_of(v, 8)


def sc_edge_agg(hs, srcc, dstc, n, npad):
    zrows = npad // NS
    nfull, rem = divmod(zrows, ECH)
    nblk = -(-CPT // IBLK)

    @functools.partial(
        pl.kernel, mesh=_mesh(), compiler_params=_CP,
        out_type=jax.ShapeDtypeStruct((NC, NS, zrows, D), jnp.float32),
        scratch_types=[
            pltpu.VMEM((IBLK, ECH), jnp.int32),
            pltpu.VMEM((IBLK, ECH), jnp.int32),
            pltpu.VMEM((ECH, D), jnp.float32),
            pltpu.VMEM((ECH, D), jnp.float32),
            pltpu.VMEM_SHARED((npad, D), jnp.float32),
            pltpu.SemaphoreType.DMA,
            pltpu.SemaphoreType.DMA,
        ],
    )
    def k(hs_h, src_h, dst_h, out_h, src_v, dst_v, rows0, rows1, acc, s0, s1):
        cid = lax.axis_index("c")
        sid = lax.axis_index("s")
        wid = sid * NC + cid
        zbase = _mo8(sid * zrows)

        def zb(i, _):
            rows0[i // (D // L),
                  pl.ds((i % (D // L)) * L, L)] = jnp.zeros((L,), jnp.float32)
            return 0

        lax.fori_loop(0, ECH * (D // L), zb, 0)
        for c in range(nfull):
            pltpu.sync_copy(rows0, acc.at[pl.ds(zbase + c * ECH, ECH)])
        if rem:
            pltpu.sync_copy(rows0.at[pl.ds(0, rem)],
                            acc.at[pl.ds(zbase + nfull * ECH, rem)])
        plsc.subcore_barrier()

        bufs = (rows0, rows1)
        sems = (s0, s1)
        for blk in range(nblk):
            bsz = min(IBLK, CPT - blk * IBLK)
            pltpu.sync_copy(src_h.at[wid, pl.ds(blk * IBLK, bsz)],
                            src_v.at[pl.ds(0, bsz)])
            pltpu.sync_copy(dst_h.at[wid, pl.ds(blk * IBLK, bsz)],
                            dst_v.at[pl.ds(0, bsz)])
            descs = [None, None]
            descs[0] = pltpu.async_copy(hs_h.at[src_v.at[0]], rows0, s0)
            for j in range(bsz):
                pbuf = j % 2
                descs[pbuf].wait()
                if j + 1 < bsz:
                    q = (j + 1) % 2
                    descs[q] = pltpu.async_copy(
                        hs_h.at[src_v.at[j + 1]], bufs[q], sems[q])
                pltpu.sync_copy(bufs[pbuf], acc.at[dst_v.at[j]], add=True)
        plsc.subcore_barrier()

        pltpu.sync_copy(acc.at[pl.ds(zbase, zrows)], out_h.at[cid, sid])

    return k(hs, srcc, dstc).reshape(NC, npad, D)


def _deg_combine(dcount, sdeg, tmp, accv, deg_h, cid, sid, npad):
    spt = npad // NS
    pltpu.sync_copy(dcount, sdeg.at[sid])
    plsc.subcore_barrier()

    def zb(i, _):
        accv[0, pl.ds(i * L, L)] = jnp.zeros((L,), jnp.float32)
        return 0

    lax.fori_loop(0, spt // L, zb, 0)
    for r in range(NS):
        pltpu.sync_copy(sdeg.at[r, sid], tmp)

        def addb(i, _):
            accv[0, pl.ds(i * L, L)] = (accv[0, pl.ds(i * L, L)]
                                        + tmp[0, pl.ds(i * L, L)])
            return 0

        lax.fori_loop(0, spt // L, addb, 0)
    pltpu.sync_copy(accv, deg_h.at[cid, sid])


def _zero_dcount(dcount, spt):
    def zb(i, _):
        dcount[i // (spt // L), 0,
               pl.ds((i % (spt // L)) * L, L)] = jnp.zeros((L,), jnp.float32)
        return 0

    lax.fori_loop(0, NS * (spt // L), zb, 0)


def _count_into(dcount, idx, spt):
    cnt, last = plsc.scan_count(idx)
    hi = idx // spt
    lo = idx - hi * spt
    zero = jnp.zeros((L,), jnp.int32)
    plsc.addupdate_scatter(dcount, [hi, zero, lo], cnt.astype(jnp.float32),
                           mask=last)


def sc_deg(dstc, npad):
    spt = npad // NS

    @functools.partial(
        pl.kernel, mesh=_mesh(), compiler_params=_CP,
        out_type=jax.ShapeDtypeStruct((NC, NS, 1, spt), jnp.float32),
        scratch_types=[
            pltpu.VMEM((CPT, ECH), jnp.int32),
            pltpu.VMEM((NS, 1, spt), jnp.float32),
            pltpu.VMEM((1, spt), jnp.float32),
            pltpu.VMEM((1, spt), jnp.float32),
            pltpu.VMEM_SHARED((NS, NS, 1, spt), jnp.float32),
            pltpu.SemaphoreType.DMA,
        ],
    )
    def k(dst_h, deg_h, dst_v, dcount, tmp, accv, sdeg, sem):
        cid = lax.axis_index("c")
        sid = lax.axis_index("s")
        wid = sid * NC + cid
        _zero_dcount(dcount, spt)
        pltpu.sync_copy(dst_h.at[wid], dst_v)

        def body(c, _):
            for j in range(ECH // L):
                _count_into(dcount, dst_v[c, pl.ds(j * L, L)], spt)
            return 0

        lax.fori_loop(0, CPT, body, 0)
        _deg_combine(dcount, sdeg, tmp, accv, deg_h, cid, sid, npad)

    return k(dstc)


def sc_relabel(mapping, srcc, dstc, n_old, n_new, npad_new):
    spt = npad_new // NS

    @functools.partial(
        pl.kernel, mesh=_mesh(), compiler_params=_CP,
        out_type=[
            jax.ShapeDtypeStruct((NW, CPT, ECH), jnp.int32),
            jax.ShapeDtypeStruct((NW, CPT, ECH), jnp.int32),
            jax.ShapeDtypeStruct((NC, NS, 1, spt), jnp.float32),
        ],
        scratch_types=[
            pltpu.VMEM((n_old + L,), jnp.int32),
            pltpu.VMEM((CPT, ECH), jnp.int32),
            pltpu.VMEM((CPT, ECH), jnp.int32),
            pltpu.VMEM((CPT, ECH), jnp.int32),
            pltpu.VMEM((CPT, ECH), jnp.int32),
            pltpu.VMEM((NS, 1, spt), jnp.float32),
            pltpu.VMEM((1, spt), jnp.float32),
            pltpu.VMEM((1, spt), jnp.float32),
            pltpu.VMEM_SHARED((NS, NS, 1, spt), jnp.float32),
            pltpu.SemaphoreType.DMA,
        ],
    )
    def k(map_h, src_h, dst_h, src2_h, dst2_h, deg_h,
          map_v, src_v, dst_v, src2_v, dst2_v, dcount, tmp, accv, sdeg, sem):
        cid = lax.axis_index("c")
        sid = lax.axis_index("s")
        wid = sid * NC + cid
        _zero_dcount(dcount, spt)
        pltpu.sync_copy(map_h, map_v)
        pltpu.sync_copy(src_h.at[wid], src_v)
        pltpu.sync_copy(dst_h.at[wid], dst_v)

        zero = jnp.zeros((L,), jnp.int32)
        trash = jnp.full((L,), n_new, jnp.int32)

        def body(c, _):
            for j in range(ECH // L):
                s = src_v[c, pl.ds(j * L, L)]
                d = dst_v[c, pl.ds(j * L, L)]
                ns = plsc.load_gather(map_v, [s])
                nd = plsc.load_gather(map_v, [d])
                valid = (ns >= 0) & (nd >= 0)
                d2 = jnp.where(valid, nd, trash)
                src2_v[c, pl.ds(j * L, L)] = jnp.where(valid, ns, zero)
                dst2_v[c, pl.ds(j * L, L)] = d2
                _count_into(dcount, d2, spt)
            return 0

        lax.fori_loop(0, CPT, body, 0)
        pltpu.sync_copy(src2_v, src2_h.at[wid])
        pltpu.sync_copy(dst2_v, dst2_h.at[wid])
        _deg_combine(dcount, sdeg, tmp, accv, deg_h, cid, sid, npad_new)

    return k(mapping, srcc, dstc)


def sc_topk(probs, n, kk):
    nv = n // L
    UNR = 4
    nvu, nvrem = divmod(nv, UNR)

    @functools.partial(
        pl.kernel, mesh=_mesh(), compiler_params=_CP,
        out_type=[
            jax.ShapeDtypeStruct((kk,), jnp.int32),
            jax.ShapeDtypeStruct((kk,), jnp.float32),
            jax.ShapeDtypeStruct((n + L,), jnp.int32),
        ],
        scratch_types=[
            pltpu.VMEM((n,), jnp.float32),
            pltpu.VMEM((kk,), jnp.int32),
            pltpu.VMEM((kk,), jnp.float32),
            pltpu.VMEM((n + L,), jnp.int32),
            pltpu.SemaphoreType.DMA,
        ],
    )
    def k(p_h, sel_h, sv_h, map_h, pv, sel_v, sv_v, map_v, sem):
        cid = lax.axis_index("c")
        sid = lax.axis_index("s")

        @pl.when((cid == 0) & (sid == 0))
        def _():
            pltpu.sync_copy(p_h, pv)
            kvec = jnp.full((L,), kk, jnp.int32)
            one = jnp.ones((L,), jnp.int32)
            zero = jnp.zeros((L,), jnp.int32)

            def lane_count(pred):
                def cbody(j, accs):
                    a0, a1 = accs
                    for u in range(UNR):
                        uv = plsc.bitcast(
                            pv[pl.ds((j * UNR + u) * L, L)], jnp.int32)
                        x = jnp.where(pred(uv), one, zero)
                        if u % 2 == 0:
                            a0 = a0 + x
                        else:
                            a1 = a1 + x
                    return a0, a1

                a0, a1 = lax.fori_loop(0, nvu, cbody, (zero, zero))
                for u in range(nvrem):
                    uv = plsc.bitcast(
                        pv[pl.ds((nvu * UNR + u) * L, L)], jnp.int32)
                    a0 = a0 + jnp.where(pred(uv), one, zero)
                tot = jnp.sum(a0 + a1)
                return jnp.broadcast_to(tot, (L,))

            def sround(i, thr):
                b = jnp.broadcast_to(30 - i, (L,)).astype(jnp.int32)
                cand = thr | (one << b)
                cnt = lane_count(lambda uv: uv >= cand)
                return jnp.where(cnt >= kvec, cand, thr)

            thr = lax.fori_loop(0, 31, sround, zero)
            g_cnt = lane_count(lambda uv: uv > thr)
            need_eq = kvec - g_cnt

            def mbody(i, _):
                map_v[pl.ds(i * L, L)] = jnp.full((L,), -1, jnp.int32)
                return 0

            lax.fori_loop(0, nv + 1, mbody, 0)
            iota = _IOTA()

            def sbody(j, carry):
                off, eqs = carry
                v = pv[pl.ds(j * L, L)]
                u = plsc.bitcast(v, jnp.int32)
                m_gt = u > thr
                m_eq = u == thr
                eqpos = plsc.cumsum(jnp.where(m_eq, one, zero))
                m = m_gt | (m_eq & ((eqs + eqpos) <= need_eq))
                rank = off + plsc.cumsum(jnp.where(m, one, zero)) - one
                idxv = jnp.full((L,), j * L, jnp.int32) + iota
                plsc.store_scatter(sel_v, [rank], idxv, mask=m)
                plsc.store_scatter(sv_v, [rank], v, mask=m)
                plsc.store_scatter(map_v, [idxv], rank, mask=m)
                off = off + plsc.all_reduce_population_count(m)
                eqs = eqs + plsc.all_reduce_population_count(m_eq)
                return off, eqs

            lax.fori_loop(0, nv, sbody, (zero, zero))
            pltpu.sync_copy(sel_v, sel_h)
            pltpu.sync_copy(sv_v, sv_h)
            pltpu.sync_copy(map_v, map_h)

    return k(probs)


GCH = 80


def sc_gather(tbl, gidx, nout):
    nch = nout // GCH
    iters = -(-nch // NW)

    @functools.partial(
        pl.kernel, mesh=_mesh(), compiler_params=_CP,
        out_type=jax.ShapeDtypeStruct((nch, GCH, D), jnp.float32),
        scratch_types=[
            pltpu.VMEM((1, GCH), jnp.int32),
            pltpu.VMEM((GCH, D), jnp.float32),
            pltpu.SemaphoreType.DMA,
        ],
    )
    def k(tbl_h, gidx_h, out_h, idx_v, rows_v, sem):
        cid = lax.axis_index("c")
        sid = lax.axis_index("s")
        wid = sid * NC + cid
        for it in range(iters):
            c = it * NW + wid

            @pl.when(c < nch)
            def _():
                pltpu.sync_copy(gidx_h.at[c], idx_v)
                pltpu.async_copy(tbl_h.at[idx_v.at[0]], rows_v, sem).wait()
                pltpu.sync_copy(rows_v, out_h.at[c])

    return k(tbl, gidx)


def _inv_norm(dref):
    return lax.rsqrt(1.0 + dref[0] + dref[1])


def _row_call(body, n, blk, outs, ins):
    grid = (n // blk,)

    def spec(a, kind):
        if kind == 'r':
            return pl.BlockSpec((blk, a.shape[1]), lambda i: (i, 0))
        if kind in ('d', 'p'):
            return pl.BlockSpec((2, blk, a.shape[2]), lambda i: (0, i, 0))
        return pl.BlockSpec(a.shape, lambda i: tuple(0 for _ in a.shape))

    return pl.pallas_call(
        body,
        grid=grid,
        in_specs=[spec(a, kd) for a, kd in ins],
        out_specs=[spec(a, kd) for a, kd in outs],
        out_shape=[jax.ShapeDtypeStruct(a.shape, a.dtype) for a, _ in outs],
    )(*[a for a, _ in ins])


def _sd(shape, dtype=jnp.float32):
    return jax.ShapeDtypeStruct(shape, dtype)


def tc_mm_scale(x, W, degp, n, blk):
    def body(x_r, d_r, w_r, o_r):
        invn = _inv_norm(d_r)
        o_r[...] = jnp.dot(x_r[...], w_r[...],
                           preferred_element_type=jnp.float32) * invn

    return _row_call(body, n, blk, [(_sd((n, D)), 'r')],
                     [(x, 'r'), (degp, 'd'), (W, 'f')])[0]


def tc_a1(P, hs0, degp, b0, Wd, bd, a, Wg, Wbi, n, blk):
    def body(p_r, hs_r, d_r, b0_r, wd_r, bd_r, a_r, wg_r, wbi_r,
             h1_r, hsg_r, tb_r):
        invn = _inv_norm(d_r)
        h1 = jax.nn.relu(invn * (p_r[0] + p_r[1] + hs_r[...]) + b0_r[...])
        h1_r[...] = h1
        hd = jnp.dot(h1, wd_r[...], preferred_element_type=jnp.float32) \
            + bd_r[...]
        hd = jnp.where(hd >= 0, hd, a_r[0, 0] * hd)
        hsg_r[...] = jnp.dot(hd, wg_r[...],
                             preferred_element_type=jnp.float32) * invn
        tb_r[...] = jnp.dot(hd, wbi_r[...], preferred_element_type=jnp.float32)

    return _row_call(
        body, n, blk,
        [(_sd((n, D)), 'r'), (_sd((n, D)), 'r'), (_sd((n, D)), 'r')],
        [(P, 'p'), (hs0, 'r'), (degp, 'd'), (b0, 'f'), (Wd, 'f'),
         (bd, 'f'), (a, 'f'), (Wg, 'f'), (Wbi, 'f')])


def tc_a2(h2, degp, Wd, bd, a, Wg, Wbi, n, blk):
    def body(h_r, d_r, wd_r, bd_r, a_r, wg_r, wbi_r, hsg_r, tb_r):
        invn = _inv_norm(d_r)
        hd = jnp.dot(h_r[...], wd_r[...],
                     preferred_element_type=jnp.float32) + bd_r[...]
        hd = jnp.where(hd >= 0, hd, a_r[0, 0] * hd)
        hsg_r[...] = jnp.dot(hd, wg_r[...],
                             preferred_element_type=jnp.float32) * invn
        tb_r[...] = jnp.dot(hd, wbi_r[...], preferred_element_type=jnp.float32)

    return _row_call(
        body, n, blk,
        [(_sd((n, D)), 'r'), (_sd((n, D)), 'r')],
        [(h2, 'r'), (degp, 'd'), (Wd, 'f'), (bd, 'f'), (a, 'f'),
         (Wg, 'f'), (Wbi, 'f')])


def tc_score(Q, hsg, tb, degp, bg, bbi, n, blk):
    def body(q_r, hsg_r, tb_r, d_r, bg_r, bbi_r, o_r):
        invn = _inv_norm(d_r)
        hx = invn * (q_r[0] + q_r[1] + hsg_r[...]) + bg_r[...]
        s = jnp.sum(tb_r[...] * hx, axis=1, keepdims=True) + bbi_r[0, 0]
        o_r[...] = jax.nn.sigmoid(s)

    return _row_call(
        body, n, blk, [(_sd((n, 1)), 'r')],
        [(Q, 'p'), (hsg, 'r'), (tb, 'r'), (degp, 'd'), (bg, 'f'),
         (bbi, 'f')])[0]


def tc_epilogue_mm(P, hs, degp, b, W, n, blk):
    def body(p_r, hs_r, d_r, b_r, w_r, o_r):
        invn = _inv_norm(d_r)
        hm = jax.nn.relu(invn * (p_r[0] + p_r[1] + hs_r[...]) + b_r[...])
        o_r[...] = jnp.dot(hm, w_r[...], preferred_element_type=jnp.float32)

    return _row_call(body, n, blk, [(_sd((n, D)), 'r')],
                     [(P, 'p'), (hs, 'r'), (degp, 'd'), (b, 'f'), (W, 'f')])[0]


def tc_gs_prep(mapping, degp, n, blk):
    def body(m_r, d_r, g_r, s_r):
        invn = _inv_norm(d_r)
        m = m_r[...]
        g_r[...] = jnp.maximum(m, 0)
        s_r[...] = jnp.where(m >= 0, invn, 0.0)

    return _row_call(
        body, n, blk,
        [(_sd((n, 1), jnp.int32), 'r'), (_sd((n, 1)), 'r')],
        [(mapping, 'r'), (degp, 'd')])


def tc_scale(raw, scl, n, blk):
    def body(x_r, s_r, o_r):
        o_r[...] = x_r[...] * s_r[...]

    return _row_call(body, n, blk, [(_sd((n, D)), 'r')],
                     [(raw, 'r'), (scl, 'r')])[0]


def tc_cross(Rm, hsm, U, hsf, degp, bm, bu, W, n, blk):
    def body(rm_r, hsm_r, u_r, hsf_r, d_r, bm_r, bu_r, w_r, o_r):
        invn = _inv_norm(d_r)
        hm = jax.nn.relu(invn * (rm_r[0] + rm_r[1] + hsm_r[...]) + bm_r[...])
        hc = hm + invn * (u_r[0] + u_r[1] + hsf_r[...]) + bu_r[...]
        o_r[...] = jnp.dot(hc, w_r[...], preferred_element_type=jnp.float32)

    return _row_call(
        body, n, blk, [(_sd((n, D)), 'r')],
        [(Rm, 'p'), (hsm, 'r'), (U, 'p'), (hsf, 'r'), (degp, 'd'),
         (bm, 'f'), (bu, 'f'), (W, 'f')])[0]


def tc_cross_final(Rm, hsm, U, hsf, degp, bm, bu, h1, WeA, WeB, n, blk):
    def body(rm_r, hsm_r, u_r, hsf_r, d_r, bm_r, bu_r, h1_r, wa_r, wb_r, o_r):
        invn = _inv_norm(d_r)
        hm = jax.nn.relu(invn * (rm_r[0] + rm_r[1] + hsm_r[...]) + bm_r[...])
        hc = hm + invn * (u_r[0] + u_r[1] + hsf_r[...]) + bu_r[...]
        o_r[...] = (jnp.dot(h1_r[...], wa_r[...],
                            preferred_element_type=jnp.float32)
                    + jnp.dot(hc, wb_r[...],
                              preferred_element_type=jnp.float32)) * invn

    return _row_call(
        body, n, blk, [(_sd((n, D)), 'r')],
        [(Rm, 'p'), (hsm, 'r'), (U, 'p'), (hsf, 'r'), (degp, 'd'),
         (bm, 'f'), (bu, 'f'), (h1, 'r'), (WeA, 'f'), (WeB, 'f')])[0]


def tc_final(P, hs, degp, b, n, blk):
    def body(p_r, hs_r, d_r, b_r, o_r):
        invn = _inv_norm(d_r)
        o_r[...] = invn * (p_r[0] + p_r[1] + hs_r[...]) + b_r[...]

    return _row_call(body, n, blk, [(_sd((n, D)), 'r')],
                     [(P, 'p'), (hs, 'r'), (degp, 'd'), (b, 'f')])[0]


def kernel(x, params, edge_index):
    p = params
    src = edge_index[0]
    dst = edge_index[1]
    pad = EP - E
    srcc = jnp.concatenate([src, jnp.zeros((pad,), jnp.int32)]).reshape(
        NW, CPT, ECH)
    dstc = jnp.concatenate([dst, jnp.full((pad,), N, jnp.int32)]).reshape(
        NW, CPT, ECH)

    gs = lambda v: v.reshape(v.shape[0] // GCH, 1, GCH)
    r2 = lambda v: v.reshape(v.shape[0], 1)
    rd = lambda dg: dg.reshape(2, dg.shape[1] * dg.shape[3], 1)
    b = {k_: p[k_].reshape(1, D) for k_ in
         ('b0', 'bd1', 'bg1', 'bd2', 'bg2', 'bm1', 'bm2', 'bm3',
          'bu1', 'bu2', 'be')}
    s = {k_: p[k_].reshape(1, 1) for k_ in ('a1', 'bbi1', 'a2', 'bbi2')}

    degp1 = rd(sc_deg(dstc, NPAD1))

    hs0 = tc_mm_scale(x, p['W0'], degp1, N, 1000)
    P0 = sc_edge_agg(hs0, srcc, dstc, N, NPAD1)
    h1, hsg1, tb1 = tc_a1(P0, hs0, degp1, b['b0'], p['Wd1'], b['bd1'],
                          s['a1'], p['Wg1'], p['Wbi1'], N, 1000)
    Q1 = sc_edge_agg(hsg1, srcc, dstc, N, NPAD1)
    probs1 = tc_score(Q1, hsg1, tb1, degp1, b['bg1'], s['bbi1'], N, 1000)
    idx1, sc1, map1 = sc_topk(probs1.reshape(N), N, K1)

    srcc2, dstc2, degp2 = sc_relabel(map1, srcc, dstc, N, K1, NPAD2)
    degp2 = rd(degp2)
    h2 = tc_scale(sc_gather(h1, gs(idx1), K1).reshape(K1, D),
                  r2(sc1), K1, 1000)

    hsg2, tb2 = tc_a2(h2, degp2, p['Wd2'], b['bd2'], s['a2'], p['Wg2'],
                      p['Wbi2'], K1, 1000)
    Q2 = sc_edge_agg(hsg2, srcc2, dstc2, K1, NPAD2)
    probs2 = tc_score(Q2, hsg2, tb2, degp2, b['bg2'], s['bbi2'], K1, 1000)
    idx2, sc2, map2 = sc_topk(probs2.reshape(K1), K1, K2)

    srcc3, dstc3, degp3 = sc_relabel(map2, srcc2, dstc2, K1, K2, NPAD3)
    degp3 = rd(degp3)
    h3 = tc_scale(sc_gather(h2, gs(idx2), K2).reshape(K2, D),
                  r2(sc2), K2, 800)

    hsm1 = tc_mm_scale(h1, p['Wm1'], degp1, N, 1000)
    R1 = sc_edge_agg(hsm1, srcc, dstc, N, NPAD1)
    hsm2 = tc_mm_scale(h2, p['Wm2'], degp2, K1, 1000)
    R2 = sc_edge_agg(hsm2, srcc2, dstc2, K1, NPAD2)
    hsm3 = tc_mm_scale(h3, p['Wm3'], degp3, K2, 800)
    R3 = sc_edge_agg(hsm3, srcc3, dstc3, K2, NPAD3)

    fw3 = tc_epilogue_mm(R3, hsm3, degp3, b['bm3'], p['Wu2'], K2, 800)
    gidx2, scl2 = tc_gs_prep(r2(map2[:K1]), degp2, K1, 1000)
    hsf2 = tc_scale(sc_gather(fw3, gs(gidx2.reshape(K1)), K1).reshape(K1, D),
                    scl2, K1, 1000)
    U2 = sc_edge_agg(hsf2, srcc2, dstc2, K1, NPAD2)

    fw2 = tc_cross(R2, hsm2, U2, hsf2, degp2, b['bm2'], b['bu2'],
                   p['Wu1'], K1, 1000)
    gidx1, scl1 = tc_gs_prep(r2(map1[:N]), degp1, N, 1000)
    hsf1 = tc_scale(sc_gather(fw2, gs(gidx1.reshape(N)), N).reshape(N, D),
                    scl1, N, 1000)
    U1 = sc_edge_agg(hsf1, srcc, dstc, N, NPAD1)

    hse = tc_cross_final(R1, hsm1, U1, hsf1, degp1, b['bm1'], b['bu1'],
                         h1, p['We'][:D], p['We'][D:], N, 1000)
    Pe = sc_edge_agg(hse, srcc, dstc, N, NPAD1)
    out = tc_final(Pe, hse, degp1, b['be'], N, 1000)
    return out


# speedup vs baseline: 1.0011x; 1.0011x over previous
import functools
import math

import jax
import jax.numpy as jnp
from jax import lax
from jax.experimental import pallas as pl
from jax.experimental.pallas import tpu as pltpu
from jax.experimental.pallas import tpu_sc as plsc

N = 10000
E = 320000
D = 128
K1 = int(math.ceil(N * 0.8))
K2 = int(math.ceil(K1 * 0.7))

NC, NS, L = 2, 16, 16
NW = NC * NS

ECH = 128
NCHUNK_RAW = -(-E // ECH)
CPT = -(-NCHUNK_RAW // NW)
NCHUNK = CPT * NW
EP = NCHUNK * ECH
IBLK = 16

NPAD1 = 10112
NPAD2 = 8064
NPAD3 = 5632

_CP = pltpu.CompilerParams(needs_layout_passes=False)

_mesh = lambda: plsc.VectorSubcoreMesh(core_axis_name="c", subcore_axis_name="s")

_IOTA = lambda: lax.iota(jnp.int32, L)


def _mo8(v):
    return pl.multiple_of(v, 8)


def sc_edge_agg(hs, srcc, dstc, n, npad):
    zrows = npad // NS
    nfull, rem = divmod(zrows, ECH)
    nblk = -(-CPT // IBLK)

    @functools.partial(
        pl.kernel, mesh=_mesh(), compiler_params=_CP,
        out_type=jax.ShapeDtypeStruct((NC, NS, zrows, D), jnp.float32),
        scratch_types=[
            pltpu.VMEM((IBLK, ECH), jnp.int32),
            pltpu.VMEM((IBLK, ECH), jnp.int32),
            pltpu.VMEM((ECH, D), jnp.float32),
            pltpu.VMEM((ECH, D), jnp.float32),
            pltpu.VMEM_SHARED((npad, D), jnp.float32),
            pltpu.SemaphoreType.DMA,
            pltpu.SemaphoreType.DMA,
        ],
    )
    def k(hs_h, src_h, dst_h, out_h, src_v, dst_v, rows0, rows1, acc, s0, s1):
        cid = lax.axis_index("c")
        sid = lax.axis_index("s")
        wid = sid * NC + cid
        zbase = _mo8(sid * zrows)

        def zb(i, _):
            rows0[i // (D // L),
                  pl.ds((i % (D // L)) * L, L)] = jnp.zeros((L,), jnp.float32)
            return 0

        lax.fori_loop(0, ECH * (D // L), zb, 0)
        for c in range(nfull):
            pltpu.sync_copy(rows0, acc.at[pl.ds(zbase + c * ECH, ECH)])
        if rem:
            pltpu.sync_copy(rows0.at[pl.ds(0, rem)],
                            acc.at[pl.ds(zbase + nfull * ECH, rem)])
        plsc.subcore_barrier()

        bufs = (rows0, rows1)
        sems = (s0, s1)
        for blk in range(nblk):
            bsz = min(IBLK, CPT - blk * IBLK)
            pltpu.sync_copy(src_h.at[wid, pl.ds(blk * IBLK, bsz)],
                            src_v.at[pl.ds(0, bsz)])
            pltpu.sync_copy(dst_h.at[wid, pl.ds(blk * IBLK, bsz)],
                            dst_v.at[pl.ds(0, bsz)])
            descs = [None, None]
            descs[0] = pltpu.async_copy(hs_h.at[src_v.at[0]], rows0, s0)
            for j in range(bsz):
                pbuf = j % 2
                descs[pbuf].wait()
                if j + 1 < bsz:
                    q = (j + 1) % 2
                    descs[q] = pltpu.async_copy(
                        hs_h.at[src_v.at[j + 1]], bufs[q], sems[q])
                pltpu.sync_copy(bufs[pbuf], acc.at[dst_v.at[j]], add=True)
        plsc.subcore_barrier()

        pltpu.sync_copy(acc.at[pl.ds(zbase, zrows)], out_h.at[cid, sid])

    return k(hs, srcc, dstc).reshape(NC, npad, D)


def _deg_combine(dcount, sdeg, tmp, accv, deg_h, cid, sid, npad):
    spt = npad // NS
    pltpu.sync_copy(dcount, sdeg.at[sid])
    plsc.subcore_barrier()

    def zb(i, _):
        accv[0, pl.ds(i * L, L)] = jnp.zeros((L,), jnp.float32)
        return 0

    lax.fori_loop(0, spt // L, zb, 0)
    for r in range(NS):
        pltpu.sync_copy(sdeg.at[r, sid], tmp)

        def addb(i, _):
            accv[0, pl.ds(i * L, L)] = (accv[0, pl.ds(i * L, L)]
                                        + tmp[0, pl.ds(i * L, L)])
            return 0

        lax.fori_loop(0, spt // L, addb, 0)
    pltpu.sync_copy(accv, deg_h.at[cid, sid])


def _zero_dcount(dcount, spt):
    def zb(i, _):
        dcount[i // (spt // L), 0,
               pl.ds((i % (spt // L)) * L, L)] = jnp.zeros((L,), jnp.float32)
        return 0

    lax.fori_loop(0, NS * (spt // L), zb, 0)


def _count_into(dcount, idx, spt):
    cnt, last = plsc.scan_count(idx)
    hi = idx // spt
    lo = idx - hi * spt
    zero = jnp.zeros((L,), jnp.int32)
    plsc.addupdate_scatter(dcount, [hi, zero, lo], cnt.astype(jnp.float32),
                           mask=last)


def sc_deg(dstc, npad):
    spt = npad // NS

    @functools.partial(
        pl.kernel, mesh=_mesh(), compiler_params=_CP,
        out_type=jax.ShapeDtypeStruct((NC, NS, 1, spt), jnp.float32),
        scratch_types=[
            pltpu.VMEM((CPT, ECH), jnp.int32),
            pltpu.VMEM((NS, 1, spt), jnp.float32),
            pltpu.VMEM((1, spt), jnp.float32),
            pltpu.VMEM((1, spt), jnp.float32),
            pltpu.VMEM_SHARED((NS, NS, 1, spt), jnp.float32),
            pltpu.SemaphoreType.DMA,
        ],
    )
    def k(dst_h, deg_h, dst_v, dcount, tmp, accv, sdeg, sem):
        cid = lax.axis_index("c")
        sid = lax.axis_index("s")
        wid = sid * NC + cid
        _zero_dcount(dcount, spt)
        pltpu.sync_copy(dst_h.at[wid], dst_v)

        def body(c, _):
            for j in range(ECH // L):
                _count_into(dcount, dst_v[c, pl.ds(j * L, L)], spt)
            return 0

        lax.fori_loop(0, CPT, body, 0)
        _deg_combine(dcount, sdeg, tmp, accv, deg_h, cid, sid, npad)

    return k(dstc)


def sc_relabel(mapping, srcc, dstc, n_old, n_new, npad_new):
    spt = npad_new // NS

    @functools.partial(
        pl.kernel, mesh=_mesh(), compiler_params=_CP,
        out_type=[
            jax.ShapeDtypeStruct((NW, CPT, ECH), jnp.int32),
            jax.ShapeDtypeStruct((NW, CPT, ECH), jnp.int32),
            jax.ShapeDtypeStruct((NC, NS, 1, spt), jnp.float32),
        ],
        scratch_types=[
            pltpu.VMEM((n_old + 2 * L,), jnp.int32),
            pltpu.VMEM((CPT, ECH), jnp.int32),
            pltpu.VMEM((CPT, ECH), jnp.int32),
            pltpu.VMEM((CPT, ECH), jnp.int32),
            pltpu.VMEM((CPT, ECH), jnp.int32),
            pltpu.VMEM((NS, 1, spt), jnp.float32),
            pltpu.VMEM((1, spt), jnp.float32),
            pltpu.VMEM((1, spt), jnp.float32),
            pltpu.VMEM_SHARED((NS, NS, 1, spt), jnp.float32),
            pltpu.SemaphoreType.DMA,
        ],
    )
    def k(map_h, src_h, dst_h, src2_h, dst2_h, deg_h,
          map_v, src_v, dst_v, src2_v, dst2_v, dcount, tmp, accv, sdeg, sem):
        cid = lax.axis_index("c")
        sid = lax.axis_index("s")
        wid = sid * NC + cid
        _zero_dcount(dcount, spt)
        pltpu.sync_copy(map_h, map_v)
        pltpu.sync_copy(src_h.at[wid], src_v)
        pltpu.sync_copy(dst_h.at[wid], dst_v)

        zero = jnp.zeros((L,), jnp.int32)
        trash = jnp.full((L,), n_new, jnp.int32)
        m31 = jnp.full((L,), 31, jnp.int32)

        def body(c, _):
            for j in range(ECH // L):
                s = src_v[c, pl.ds(j * L, L)]
                d = dst_v[c, pl.ds(j * L, L)]
                ns = plsc.load_gather(map_v, [s])
                nd = plsc.load_gather(map_v, [d])
                valid = (ns >= 0) & (nd >= 0)
                d2 = jnp.where(valid, nd, trash + (d & m31))
                src2_v[c, pl.ds(j * L, L)] = jnp.where(valid, ns, zero)
                dst2_v[c, pl.ds(j * L, L)] = d2
                _count_into(dcount, d2, spt)
            return 0

        lax.fori_loop(0, CPT, body, 0)
        pltpu.sync_copy(src2_v, src2_h.at[wid])
        pltpu.sync_copy(dst2_v, dst2_h.at[wid])
        _deg_combine(dcount, sdeg, tmp, accv, deg_h, cid, sid, npad_new)

    return k(mapping, srcc, dstc)


def sc_topk(probs, n, kk):
    nv = n // L
    UNR = 4
    nvu, nvrem = divmod(nv, UNR)

    @functools.partial(
        pl.kernel, mesh=_mesh(), compiler_params=_CP,
        out_type=[
            jax.ShapeDtypeStruct((kk,), jnp.int32),
            jax.ShapeDtypeStruct((kk,), jnp.float32),
            jax.ShapeDtypeStruct((n + 2 * L,), jnp.int32),
        ],
        scratch_types=[
            pltpu.VMEM((n,), jnp.float32),
            pltpu.VMEM((kk,), jnp.int32),
            pltpu.VMEM((kk,), jnp.float32),
            pltpu.VMEM((n + 2 * L,), jnp.int32),
            pltpu.SemaphoreType.DMA,
        ],
    )
    def k(p_h, sel_h, sv_h, map_h, pv, sel_v, sv_v, map_v, sem):
        cid = lax.axis_index("c")
        sid = lax.axis_index("s")

        @pl.when((cid == 0) & (sid == 0))
        def _():
            pltpu.sync_copy(p_h, pv)
            kvec = jnp.full((L,), kk, jnp.int32)
            one = jnp.ones((L,), jnp.int32)
            zero = jnp.zeros((L,), jnp.int32)

            def lane_count(pred):
                def cbody(j, accs):
                    a0, a1 = accs
                    for u in range(UNR):
                        uv = plsc.bitcast(
                            pv[pl.ds((j * UNR + u) * L, L)], jnp.int32)
                        x = jnp.where(pred(uv), one, zero)
                        if u % 2 == 0:
                            a0 = a0 + x
                        else:
                            a1 = a1 + x
                    return a0, a1

                a0, a1 = lax.fori_loop(0, nvu, cbody, (zero, zero))
                for u in range(nvrem):
                    uv = plsc.bitcast(
                        pv[pl.ds((nvu * UNR + u) * L, L)], jnp.int32)
                    a0 = a0 + jnp.where(pred(uv), one, zero)
                tot = jnp.sum(a0 + a1)
                return jnp.broadcast_to(tot, (L,))

            def sround(i, thr):
                b = jnp.broadcast_to(30 - i, (L,)).astype(jnp.int32)
                cand = thr | (one << b)
                cnt = lane_count(lambda uv: uv >= cand)
                return jnp.where(cnt >= kvec, cand, thr)

            thr = lax.fori_loop(0, 31, sround, zero)
            g_cnt = lane_count(lambda uv: uv > thr)
            need_eq = kvec - g_cnt

            def mbody(i, _):
                map_v[pl.ds(i * L, L)] = jnp.full((L,), -1, jnp.int32)
                return 0

            lax.fori_loop(0, nv + 2, mbody, 0)
            iota = _IOTA()

            def sbody(j, carry):
                off, eqs = carry
                v = pv[pl.ds(j * L, L)]
                u = plsc.bitcast(v, jnp.int32)
                m_gt = u > thr
                m_eq = u == thr
                eqpos = plsc.cumsum(jnp.where(m_eq, one, zero))
                m = m_gt | (m_eq & ((eqs + eqpos) <= need_eq))
                rank = off + plsc.cumsum(jnp.where(m, one, zero)) - one
                idxv = jnp.full((L,), j * L, jnp.int32) + iota
                plsc.store_scatter(sel_v, [rank], idxv, mask=m)
                plsc.store_scatter(sv_v, [rank], v, mask=m)
                plsc.store_scatter(map_v, [idxv], rank, mask=m)
                off = off + plsc.all_reduce_population_count(m)
                eqs = eqs + plsc.all_reduce_population_count(m_eq)
                return off, eqs

            lax.fori_loop(0, nv, sbody, (zero, zero))
            pltpu.sync_copy(sel_v, sel_h)
            pltpu.sync_copy(sv_v, sv_h)
            pltpu.sync_copy(map_v, map_h)

    return k(probs)


GCH = 80


def sc_gather(tbl, gidx, nout):
    nch = nout // GCH
    iters = -(-nch // NW)

    @functools.partial(
        pl.kernel, mesh=_mesh(), compiler_params=_CP,
        out_type=jax.ShapeDtypeStruct((nch, GCH, D), jnp.float32),
        scratch_types=[
            pltpu.VMEM((1, GCH), jnp.int32),
            pltpu.VMEM((GCH, D), jnp.float32),
            pltpu.SemaphoreType.DMA,
        ],
    )
    def k(tbl_h, gidx_h, out_h, idx_v, rows_v, sem):
        cid = lax.axis_index("c")
        sid = lax.axis_index("s")
        wid = sid * NC + cid
        for it in range(iters):
            c = it * NW + wid

            @pl.when(c < nch)
            def _():
                pltpu.sync_copy(gidx_h.at[c], idx_v)
                pltpu.async_copy(tbl_h.at[idx_v.at[0]], rows_v, sem).wait()
                pltpu.sync_copy(rows_v, out_h.at[c])

    return k(tbl, gidx)


def _inv_norm(dref):
    return lax.rsqrt(1.0 + dref[0] + dref[1])


def _row_call(body, n, blk, outs, ins):
    grid = (n // blk,)

    def spec(a, kind):
        if kind == 'r':
            return pl.BlockSpec((blk, a.shape[1]), lambda i: (i, 0))
        if kind in ('d', 'p'):
            return pl.BlockSpec((2, blk, a.shape[2]), lambda i: (0, i, 0))
        return pl.BlockSpec(a.shape, lambda i: tuple(0 for _ in a.shape))

    return pl.pallas_call(
        body,
        grid=grid,
        in_specs=[spec(a, kd) for a, kd in ins],
        out_specs=[spec(a, kd) for a, kd in outs],
        out_shape=[jax.ShapeDtypeStruct(a.shape, a.dtype) for a, _ in outs],
    )(*[a for a, _ in ins])


def _sd(shape, dtype=jnp.float32):
    return jax.ShapeDtypeStruct(shape, dtype)


def tc_mm_scale(x, W, degp, n, blk):
    def body(x_r, d_r, w_r, o_r):
        invn = _inv_norm(d_r)
        o_r[...] = jnp.dot(x_r[...], w_r[...],
                           preferred_element_type=jnp.float32) * invn

    return _row_call(body, n, blk, [(_sd((n, D)), 'r')],
                     [(x, 'r'), (degp, 'd'), (W, 'f')])[0]


def tc_a1(P, hs0, degp, b0, Wd, bd, a, Wg, Wbi, n, blk):
    def body(p_r, hs_r, d_r, b0_r, wd_r, bd_r, a_r, wg_r, wbi_r,
             h1_r, hsg_r, tb_r):
        invn = _inv_norm(d_r)
        h1 = jax.nn.relu(invn * (p_r[0] + p_r[1] + hs_r[...]) + b0_r[...])
        h1_r[...] = h1
        hd = jnp.dot(h1, wd_r[...], preferred_element_type=jnp.float32) \
            + bd_r[...]
        hd = jnp.where(hd >= 0, hd, a_r[0, 0] * hd)
        hsg_r[...] = jnp.dot(hd, wg_r[...],
                             preferred_element_type=jnp.float32) * invn
        tb_r[...] = jnp.dot(hd, wbi_r[...], preferred_element_type=jnp.float32)

    return _row_call(
        body, n, blk,
        [(_sd((n, D)), 'r'), (_sd((n, D)), 'r'), (_sd((n, D)), 'r')],
        [(P, 'p'), (hs0, 'r'), (degp, 'd'), (b0, 'f'), (Wd, 'f'),
         (bd, 'f'), (a, 'f'), (Wg, 'f'), (Wbi, 'f')])


def tc_a2(h2, degp, Wd, bd, a, Wg, Wbi, n, blk):
    def body(h_r, d_r, wd_r, bd_r, a_r, wg_r, wbi_r, hsg_r, tb_r):
        invn = _inv_norm(d_r)
        hd = jnp.dot(h_r[...], wd_r[...],
                     preferred_element_type=jnp.float32) + bd_r[...]
        hd = jnp.where(hd >= 0, hd, a_r[0, 0] * hd)
        hsg_r[...] = jnp.dot(hd, wg_r[...],
                             preferred_element_type=jnp.float32) * invn
        tb_r[...] = jnp.dot(hd, wbi_r[...], preferred_element_type=jnp.float32)

    return _row_call(
        body, n, blk,
        [(_sd((n, D)), 'r'), (_sd((n, D)), 'r')],
        [(h2, 'r'), (degp, 'd'), (Wd, 'f'), (bd, 'f'), (a, 'f'),
         (Wg, 'f'), (Wbi, 'f')])


def tc_score(Q, hsg, tb, degp, bg, bbi, n, blk):
    def body(q_r, hsg_r, tb_r, d_r, bg_r, bbi_r, o_r):
        invn = _inv_norm(d_r)
        hx = invn * (q_r[0] + q_r[1] + hsg_r[...]) + bg_r[...]
        s = jnp.sum(tb_r[...] * hx, axis=1, keepdims=True) + bbi_r[0, 0]
        o_r[...] = jax.nn.sigmoid(s)

    return _row_call(
        body, n, blk, [(_sd((n, 1)), 'r')],
        [(Q, 'p'), (hsg, 'r'), (tb, 'r'), (degp, 'd'), (bg, 'f'),
         (bbi, 'f')])[0]


def tc_epilogue_mm(P, hs, degp, b, W, n, blk):
    def body(p_r, hs_r, d_r, b_r, w_r, o_r):
        invn = _inv_norm(d_r)
        hm = jax.nn.relu(invn * (p_r[0] + p_r[1] + hs_r[...]) + b_r[...])
        o_r[...] = jnp.dot(hm, w_r[...], preferred_element_type=jnp.float32)

    return _row_call(body, n, blk, [(_sd((n, D)), 'r')],
                     [(P, 'p'), (hs, 'r'), (degp, 'd'), (b, 'f'), (W, 'f')])[0]


def tc_gs_prep(mapping, degp, n, blk):
    def body(m_r, d_r, g_r, s_r):
        invn = _inv_norm(d_r)
        m = m_r[...]
        g_r[...] = jnp.maximum(m, 0)
        s_r[...] = jnp.where(m >= 0, invn, 0.0)

    return _row_call(
        body, n, blk,
        [(_sd((n, 1), jnp.int32), 'r'), (_sd((n, 1)), 'r')],
        [(mapping, 'r'), (degp, 'd')])


def tc_scale(raw, scl, n, blk):
    def body(x_r, s_r, o_r):
        o_r[...] = x_r[...] * s_r[...]

    return _row_call(body, n, blk, [(_sd((n, D)), 'r')],
                     [(raw, 'r'), (scl, 'r')])[0]


def tc_cross(Rm, hsm, U, hsf, degp, bm, bu, W, n, blk):
    def body(rm_r, hsm_r, u_r, hsf_r, d_r, bm_r, bu_r, w_r, o_r):
        invn = _inv_norm(d_r)
        hm = jax.nn.relu(invn * (rm_r[0] + rm_r[1] + hsm_r[...]) + bm_r[...])
        hc = hm + invn * (u_r[0] + u_r[1] + hsf_r[...]) + bu_r[...]
        o_r[...] = jnp.dot(hc, w_r[...], preferred_element_type=jnp.float32)

    return _row_call(
        body, n, blk, [(_sd((n, D)), 'r')],
        [(Rm, 'p'), (hsm, 'r'), (U, 'p'), (hsf, 'r'), (degp, 'd'),
         (bm, 'f'), (bu, 'f'), (W, 'f')])[0]


def tc_cross_final(Rm, hsm, U, hsf, degp, bm, bu, h1, WeA, WeB, n, blk):
    def body(rm_r, hsm_r, u_r, hsf_r, d_r, bm_r, bu_r, h1_r, wa_r, wb_r, o_r):
        invn = _inv_norm(d_r)
        hm = jax.nn.relu(invn * (rm_r[0] + rm_r[1] + hsm_r[...]) + bm_r[...])
        hc = hm + invn * (u_r[0] + u_r[1] + hsf_r[...]) + bu_r[...]
        o_r[...] = (jnp.dot(h1_r[...], wa_r[...],
                            preferred_element_type=jnp.float32)
                    + jnp.dot(hc, wb_r[...],
                              preferred_element_type=jnp.float32)) * invn

    return _row_call(
        body, n, blk, [(_sd((n, D)), 'r')],
        [(Rm, 'p'), (hsm, 'r'), (U, 'p'), (hsf, 'r'), (degp, 'd'),
         (bm, 'f'), (bu, 'f'), (h1, 'r'), (WeA, 'f'), (WeB, 'f')])[0]


def tc_final(P, hs, degp, b, n, blk):
    def body(p_r, hs_r, d_r, b_r, o_r):
        invn = _inv_norm(d_r)
        o_r[...] = invn * (p_r[0] + p_r[1] + hs_r[...]) + b_r[...]

    return _row_call(body, n, blk, [(_sd((n, D)), 'r')],
                     [(P, 'p'), (hs, 'r'), (degp, 'd'), (b, 'f')])[0]


def kernel(x, params, edge_index):
    p = params
    src = edge_index[0]
    dst = edge_index[1]
    pad = EP - E
    srcc = jnp.concatenate([src, jnp.zeros((pad,), jnp.int32)]).reshape(
        NW, CPT, ECH)
    dstc = jnp.concatenate(
        [dst, N + (jnp.arange(pad, dtype=jnp.int32) % 32)]).reshape(
        NW, CPT, ECH)

    gs = lambda v: v.reshape(v.shape[0] // GCH, 1, GCH)
    r2 = lambda v: v.reshape(v.shape[0], 1)
    rd = lambda dg: dg.reshape(2, dg.shape[1] * dg.shape[3], 1)
    b = {k_: p[k_].reshape(1, D) for k_ in
         ('b0', 'bd1', 'bg1', 'bd2', 'bg2', 'bm1', 'bm2', 'bm3',
          'bu1', 'bu2', 'be')}
    s = {k_: p[k_].reshape(1, 1) for k_ in ('a1', 'bbi1', 'a2', 'bbi2')}

    degp1 = rd(sc_deg(dstc, NPAD1))

    hs0 = tc_mm_scale(x, p['W0'], degp1, N, 1000)
    P0 = sc_edge_agg(hs0, srcc, dstc, N, NPAD1)
    h1, hsg1, tb1 = tc_a1(P0, hs0, degp1, b['b0'], p['Wd1'], b['bd1'],
                          s['a1'], p['Wg1'], p['Wbi1'], N, 1000)
    Q1 = sc_edge_agg(hsg1, srcc, dstc, N, NPAD1)
    probs1 = tc_score(Q1, hsg1, tb1, degp1, b['bg1'], s['bbi1'], N, 1000)
    idx1, sc1, map1 = sc_topk(probs1.reshape(N), N, K1)

    srcc2, dstc2, degp2 = sc_relabel(map1, srcc, dstc, N, K1, NPAD2)
    degp2 = rd(degp2)
    h2 = tc_scale(sc_gather(h1, gs(idx1), K1).reshape(K1, D),
                  r2(sc1), K1, 1000)

    hsg2, tb2 = tc_a2(h2, degp2, p['Wd2'], b['bd2'], s['a2'], p['Wg2'],
                      p['Wbi2'], K1, 1000)
    Q2 = sc_edge_agg(hsg2, srcc2, dstc2, K1, NPAD2)
    probs2 = tc_score(Q2, hsg2, tb2, degp2, b['bg2'], s['bbi2'], K1, 1000)
    idx2, sc2, map2 = sc_topk(probs2.reshape(K1), K1, K2)

    srcc3, dstc3, degp3 = sc_relabel(map2, srcc2, dstc2, K1, K2, NPAD3)
    degp3 = rd(degp3)
    h3 = tc_scale(sc_gather(h2, gs(idx2), K2).reshape(K2, D),
                  r2(sc2), K2, 800)

    hsm1 = tc_mm_scale(h1, p['Wm1'], degp1, N, 1000)
    R1 = sc_edge_agg(hsm1, srcc, dstc, N, NPAD1)
    hsm2 = tc_mm_scale(h2, p['Wm2'], degp2, K1, 1000)
    R2 = sc_edge_agg(hsm2, srcc2, dstc2, K1, NPAD2)
    hsm3 = tc_mm_scale(h3, p['Wm3'], degp3, K2, 800)
    R3 = sc_edge_agg(hsm3, srcc3, dstc3, K2, NPAD3)

    fw3 = tc_epilogue_mm(R3, hsm3, degp3, b['bm3'], p['Wu2'], K2, 800)
    gidx2, scl2 = tc_gs_prep(r2(map2[:K1]), degp2, K1, 1000)
    hsf2 = tc_scale(sc_gather(fw3, gs(gidx2.reshape(K1)), K1).reshape(K1, D),
                    scl2, K1, 1000)
    U2 = sc_edge_agg(hsf2, srcc2, dstc2, K1, NPAD2)

    fw2 = tc_cross(R2, hsm2, U2, hsf2, degp2, b['bm2'], b['bu2'],
                   p['Wu1'], K1, 1000)
    gidx1, scl1 = tc_gs_prep(r2(map1[:N]), degp1, N, 1000)
    hsf1 = tc_scale(sc_gather(fw2, gs(gidx1.reshape(N)), N).reshape(N, D),
                    scl1, N, 1000)
    U1 = sc_edge_agg(hsf1, srcc, dstc, N, NPAD1)

    hse = tc_cross_final(R1, hsm1, U1, hsf1, degp1, b['bm1'], b['bu1'],
                         h1, p['We'][:D], p['We'][D:], N, 1000)
    Pe = sc_edge_agg(hse, srcc, dstc, N, NPAD1)
    out = tc_final(Pe, hse, degp1, b['be'], N, 1000)
    return out


# speedup vs baseline: 12.4265x; 12.4131x over previous
import functools
import math

import jax
import jax.numpy as jnp
from jax import lax
from jax.experimental import pallas as pl
from jax.experimental.pallas import tpu as pltpu
from jax.experimental.pallas import tpu_sc as plsc

N = 10000
E = 320000
D = 128
K1 = int(math.ceil(N * 0.8))
K2 = int(math.ceil(K1 * 0.7))

NC, NS, L = 2, 16, 16
NW = NC * NS

ECH = 128
NCHUNK_RAW = -(-E // ECH)
CPT = -(-NCHUNK_RAW // NW)
NCHUNK = CPT * NW
EP = NCHUNK * ECH
IBLK = 16

NPAD1 = 10112
NPAD2 = 8064
NPAD3 = 5632

_CP = pltpu.CompilerParams(needs_layout_passes=False)

_mesh = lambda: plsc.VectorSubcoreMesh(core_axis_name="c", subcore_axis_name="s")

_IOTA = lambda: lax.iota(jnp.int32, L)


def _mo8(v):
    return pl.multiple_of(v, 8)


def sc_edge_agg(hs, srcc, dstc, n, npad):
    zrows = npad // NS
    nfull, rem = divmod(zrows, ECH)
    nblk = -(-CPT // IBLK)

    @functools.partial(
        pl.kernel, mesh=_mesh(), compiler_params=_CP,
        out_type=jax.ShapeDtypeStruct((NC, NS, zrows, D), jnp.float32),
        scratch_types=[
            pltpu.VMEM((IBLK, ECH), jnp.int32),
            pltpu.VMEM((IBLK, ECH), jnp.int32),
            pltpu.VMEM((ECH, D), jnp.float32),
            pltpu.VMEM((ECH, D), jnp.float32),
            pltpu.VMEM_SHARED((npad, D), jnp.float32),
            pltpu.SemaphoreType.DMA,
            pltpu.SemaphoreType.DMA,
        ],
    )
    def k(hs_h, src_h, dst_h, out_h, src_v, dst_v, rows0, rows1, acc, s0, s1):
        cid = lax.axis_index("c")
        sid = lax.axis_index("s")
        wid = sid * NC + cid
        zbase = _mo8(sid * zrows)

        def zb(i, _):
            rows0[i // (D // L),
                  pl.ds((i % (D // L)) * L, L)] = jnp.zeros((L,), jnp.float32)
            return 0

        lax.fori_loop(0, ECH * (D // L), zb, 0)
        for c in range(nfull):
            pltpu.sync_copy(rows0, acc.at[pl.ds(zbase + c * ECH, ECH)])
        if rem:
            pltpu.sync_copy(rows0.at[pl.ds(0, rem)],
                            acc.at[pl.ds(zbase + nfull * ECH, rem)])
        plsc.subcore_barrier()

        bufs = (rows0, rows1)
        sems = (s0, s1)
        for blk in range(nblk):
            bsz = min(IBLK, CPT - blk * IBLK)
            pltpu.sync_copy(src_h.at[wid, pl.ds(blk * IBLK, bsz)],
                            src_v.at[pl.ds(0, bsz)])
            pltpu.sync_copy(dst_h.at[wid, pl.ds(blk * IBLK, bsz)],
                            dst_v.at[pl.ds(0, bsz)])
            descs = [None, None]
            descs[0] = pltpu.async_copy(hs_h.at[src_v.at[0]], rows0, s0)
            for j in range(bsz):
                pbuf = j % 2
                descs[pbuf].wait()
                if j + 1 < bsz:
                    q = (j + 1) % 2
                    descs[q] = pltpu.async_copy(
                        hs_h.at[src_v.at[j + 1]], bufs[q], sems[q])
                pltpu.sync_copy(bufs[pbuf], acc.at[dst_v.at[j]], add=True)
        plsc.subcore_barrier()

        pltpu.sync_copy(acc.at[pl.ds(zbase, zrows)], out_h.at[cid, sid])

    return k(hs, srcc, dstc).reshape(NC, npad, D)


def _deg_combine(dcount, sdeg, tmp, accv, deg_h, cid, sid, npad):
    spt = npad // NS
    pltpu.sync_copy(dcount, sdeg.at[sid])
    plsc.subcore_barrier()

    def zb(i, _):
        accv[0, pl.ds(i * L, L)] = jnp.zeros((L,), jnp.float32)
        return 0

    lax.fori_loop(0, spt // L, zb, 0)
    for r in range(NS):
        pltpu.sync_copy(sdeg.at[r, sid], tmp)

        def addb(i, _):
            accv[0, pl.ds(i * L, L)] = (accv[0, pl.ds(i * L, L)]
                                        + tmp[0, pl.ds(i * L, L)])
            return 0

        lax.fori_loop(0, spt // L, addb, 0)
    pltpu.sync_copy(accv, deg_h.at[cid, sid])


def _zero_dcount(dcount, spt):
    def zb(i, _):
        dcount[i // (spt // L), 0,
               pl.ds((i % (spt // L)) * L, L)] = jnp.zeros((L,), jnp.float32)
        return 0

    lax.fori_loop(0, NS * (spt // L), zb, 0)


def _count_into(dcount, idx, spt):
    cnt, last = plsc.scan_count(idx)
    hi = idx // spt
    lo = idx - hi * spt
    zero = jnp.zeros((L,), jnp.int32)
    plsc.addupdate_scatter(dcount, [hi, zero, lo], cnt.astype(jnp.float32),
                           mask=last)


def sc_deg(dstc, npad):
    spt = npad // NS

    @functools.partial(
        pl.kernel, mesh=_mesh(), compiler_params=_CP,
        out_type=jax.ShapeDtypeStruct((NC, NS, 1, spt), jnp.float32),
        scratch_types=[
            pltpu.VMEM((CPT, ECH), jnp.int32),
            pltpu.VMEM((NS, 1, spt), jnp.float32),
            pltpu.VMEM((1, spt), jnp.float32),
            pltpu.VMEM((1, spt), jnp.float32),
            pltpu.VMEM_SHARED((NS, NS, 1, spt), jnp.float32),
            pltpu.SemaphoreType.DMA,
        ],
    )
    def k(dst_h, deg_h, dst_v, dcount, tmp, accv, sdeg, sem):
        cid = lax.axis_index("c")
        sid = lax.axis_index("s")
        wid = sid * NC + cid
        _zero_dcount(dcount, spt)
        pltpu.sync_copy(dst_h.at[wid], dst_v)

        def body(c, _):
            for j in range(ECH // L):
                _count_into(dcount, dst_v[c, pl.ds(j * L, L)], spt)
            return 0

        lax.fori_loop(0, CPT, body, 0)
        _deg_combine(dcount, sdeg, tmp, accv, deg_h, cid, sid, npad)

    return k(dstc)


def sc_relabel(mapping, srcc, dstc, n_old, n_new, npad_new):
    spt = npad_new // NS

    @functools.partial(
        pl.kernel, mesh=_mesh(), compiler_params=_CP,
        out_type=[
            jax.ShapeDtypeStruct((NW, CPT, ECH), jnp.int32),
            jax.ShapeDtypeStruct((NW, CPT, ECH), jnp.int32),
            jax.ShapeDtypeStruct((NC, NS, 1, spt), jnp.float32),
        ],
        scratch_types=[
            pltpu.VMEM((n_old + 2 * L,), jnp.int32),
            pltpu.VMEM((CPT, ECH), jnp.int32),
            pltpu.VMEM((CPT, ECH), jnp.int32),
            pltpu.VMEM((CPT, ECH), jnp.int32),
            pltpu.VMEM((CPT, ECH), jnp.int32),
            pltpu.VMEM((NS, 1, spt), jnp.float32),
            pltpu.VMEM((1, spt), jnp.float32),
            pltpu.VMEM((1, spt), jnp.float32),
            pltpu.VMEM_SHARED((NS, NS, 1, spt), jnp.float32),
            pltpu.SemaphoreType.DMA,
        ],
    )
    def k(map_h, src_h, dst_h, src2_h, dst2_h, deg_h,
          map_v, src_v, dst_v, src2_v, dst2_v, dcount, tmp, accv, sdeg, sem):
        cid = lax.axis_index("c")
        sid = lax.axis_index("s")
        wid = sid * NC + cid
        _zero_dcount(dcount, spt)
        pltpu.sync_copy(map_h, map_v)
        pltpu.sync_copy(src_h.at[wid], src_v)
        pltpu.sync_copy(dst_h.at[wid], dst_v)

        zero = jnp.zeros((L,), jnp.int32)
        trash = jnp.full((L,), n_new, jnp.int32)
        m31 = jnp.full((L,), 31, jnp.int32)
        m4095 = jnp.full((L,), 4095, jnp.int32)

        def body(c, _):
            for j in range(ECH // L):
                s = src_v[c, pl.ds(j * L, L)]
                d = dst_v[c, pl.ds(j * L, L)]
                ns = plsc.load_gather(map_v, [s])
                nd = plsc.load_gather(map_v, [d])
                valid = (ns >= 0) & (nd >= 0)
                d2 = jnp.where(valid, nd, trash + (d & m31))
                src2_v[c, pl.ds(j * L, L)] = jnp.where(valid, ns, d & m4095)
                dst2_v[c, pl.ds(j * L, L)] = d2
                _count_into(dcount, d2, spt)
            return 0

        lax.fori_loop(0, CPT, body, 0)
        pltpu.sync_copy(src2_v, src2_h.at[wid])
        pltpu.sync_copy(dst2_v, dst2_h.at[wid])
        _deg_combine(dcount, sdeg, tmp, accv, deg_h, cid, sid, npad_new)

    return k(mapping, srcc, dstc)


def sc_topk(probs, n, kk):
    nv = n // L
    UNR = 4
    nvu, nvrem = divmod(nv, UNR)

    @functools.partial(
        pl.kernel, mesh=_mesh(), compiler_params=_CP,
        out_type=[
            jax.ShapeDtypeStruct((kk,), jnp.int32),
            jax.ShapeDtypeStruct((kk,), jnp.float32),
            jax.ShapeDtypeStruct((n + 2 * L,), jnp.int32),
        ],
        scratch_types=[
            pltpu.VMEM((n,), jnp.float32),
            pltpu.VMEM((kk,), jnp.int32),
            pltpu.VMEM((kk,), jnp.float32),
            pltpu.VMEM((n + 2 * L,), jnp.int32),
            pltpu.SemaphoreType.DMA,
        ],
    )
    def k(p_h, sel_h, sv_h, map_h, pv, sel_v, sv_v, map_v, sem):
        cid = lax.axis_index("c")
        sid = lax.axis_index("s")

        @pl.when((cid == 0) & (sid == 0))
        def _():
            pltpu.sync_copy(p_h, pv)
            kvec = jnp.full((L,), kk, jnp.int32)
            one = jnp.ones((L,), jnp.int32)
            zero = jnp.zeros((L,), jnp.int32)

            def lane_count(pred):
                def cbody(j, accs):
                    a0, a1 = accs
                    for u in range(UNR):
                        uv = plsc.bitcast(
                            pv[pl.ds((j * UNR + u) * L, L)], jnp.int32)
                        x = jnp.where(pred(uv), one, zero)
                        if u % 2 == 0:
                            a0 = a0 + x
                        else:
                            a1 = a1 + x
                    return a0, a1

                a0, a1 = lax.fori_loop(0, nvu, cbody, (zero, zero))
                for u in range(nvrem):
                    uv = plsc.bitcast(
                        pv[pl.ds((nvu * UNR + u) * L, L)], jnp.int32)
                    a0 = a0 + jnp.where(pred(uv), one, zero)
                tot = jnp.sum(a0 + a1)
                return jnp.broadcast_to(tot, (L,))

            def sround(i, thr):
                b = jnp.broadcast_to(30 - i, (L,)).astype(jnp.int32)
                cand = thr | (one << b)
                cnt = lane_count(lambda uv: uv >= cand)
                return jnp.where(cnt >= kvec, cand, thr)

            thr = lax.fori_loop(0, 31, sround, zero)
            g_cnt = lane_count(lambda uv: uv > thr)
            need_eq = kvec - g_cnt

            def mbody(i, _):
                map_v[pl.ds(i * L, L)] = jnp.full((L,), -1, jnp.int32)
                return 0

            lax.fori_loop(0, nv + 2, mbody, 0)
            iota = _IOTA()

            def sbody(j, carry):
                off, eqs = carry
                v = pv[pl.ds(j * L, L)]
                u = plsc.bitcast(v, jnp.int32)
                m_gt = u > thr
                m_eq = u == thr
                eqpos = plsc.cumsum(jnp.where(m_eq, one, zero))
                m = m_gt | (m_eq & ((eqs + eqpos) <= need_eq))
                rank = off + plsc.cumsum(jnp.where(m, one, zero)) - one
                idxv = jnp.full((L,), j * L, jnp.int32) + iota
                plsc.store_scatter(sel_v, [rank], idxv, mask=m)
                plsc.store_scatter(sv_v, [rank], v, mask=m)
                plsc.store_scatter(map_v, [idxv], rank, mask=m)
                off = off + plsc.all_reduce_population_count(m)
                eqs = eqs + plsc.all_reduce_population_count(m_eq)
                return off, eqs

            lax.fori_loop(0, nv, sbody, (zero, zero))
            pltpu.sync_copy(sel_v, sel_h)
            pltpu.sync_copy(sv_v, sv_h)
            pltpu.sync_copy(map_v, map_h)

    return k(probs)


GCH = 80


def sc_gather(tbl, gidx, nout):
    nch = nout // GCH
    iters = -(-nch // NW)

    @functools.partial(
        pl.kernel, mesh=_mesh(), compiler_params=_CP,
        out_type=jax.ShapeDtypeStruct((nch, GCH, D), jnp.float32),
        scratch_types=[
            pltpu.VMEM((1, GCH), jnp.int32),
            pltpu.VMEM((GCH, D), jnp.float32),
            pltpu.SemaphoreType.DMA,
        ],
    )
    def k(tbl_h, gidx_h, out_h, idx_v, rows_v, sem):
        cid = lax.axis_index("c")
        sid = lax.axis_index("s")
        wid = sid * NC + cid
        for it in range(iters):
            c = it * NW + wid

            @pl.when(c < nch)
            def _():
                pltpu.sync_copy(gidx_h.at[c], idx_v)
                pltpu.async_copy(tbl_h.at[idx_v.at[0]], rows_v, sem).wait()
                pltpu.sync_copy(rows_v, out_h.at[c])

    return k(tbl, gidx)


def _inv_norm(dref):
    return lax.rsqrt(1.0 + dref[0] + dref[1])


def _row_call(body, n, blk, outs, ins):
    grid = (n // blk,)

    def spec(a, kind):
        if kind == 'r':
            return pl.BlockSpec((blk, a.shape[1]), lambda i: (i, 0))
        if kind in ('d', 'p'):
            return pl.BlockSpec((2, blk, a.shape[2]), lambda i: (0, i, 0))
        return pl.BlockSpec(a.shape, lambda i: tuple(0 for _ in a.shape))

    return pl.pallas_call(
        body,
        grid=grid,
        in_specs=[spec(a, kd) for a, kd in ins],
        out_specs=[spec(a, kd) for a, kd in outs],
        out_shape=[jax.ShapeDtypeStruct(a.shape, a.dtype) for a, _ in outs],
    )(*[a for a, _ in ins])


def _sd(shape, dtype=jnp.float32):
    return jax.ShapeDtypeStruct(shape, dtype)


def tc_mm_scale(x, W, degp, n, blk):
    def body(x_r, d_r, w_r, o_r):
        invn = _inv_norm(d_r)
        o_r[...] = jnp.dot(x_r[...], w_r[...],
                           preferred_element_type=jnp.float32) * invn

    return _row_call(body, n, blk, [(_sd((n, D)), 'r')],
                     [(x, 'r'), (degp, 'd'), (W, 'f')])[0]


def tc_a1(P, hs0, degp, b0, Wd, bd, a, Wg, Wbi, n, blk):
    def body(p_r, hs_r, d_r, b0_r, wd_r, bd_r, a_r, wg_r, wbi_r,
             h1_r, hsg_r, tb_r):
        invn = _inv_norm(d_r)
        h1 = jax.nn.relu(invn * (p_r[0] + p_r[1] + hs_r[...]) + b0_r[...])
        h1_r[...] = h1
        hd = jnp.dot(h1, wd_r[...], preferred_element_type=jnp.float32) \
            + bd_r[...]
        hd = jnp.where(hd >= 0, hd, a_r[0, 0] * hd)
        hsg_r[...] = jnp.dot(hd, wg_r[...],
                             preferred_element_type=jnp.float32) * invn
        tb_r[...] = jnp.dot(hd, wbi_r[...], preferred_element_type=jnp.float32)

    return _row_call(
        body, n, blk,
        [(_sd((n, D)), 'r'), (_sd((n, D)), 'r'), (_sd((n, D)), 'r')],
        [(P, 'p'), (hs0, 'r'), (degp, 'd'), (b0, 'f'), (Wd, 'f'),
         (bd, 'f'), (a, 'f'), (Wg, 'f'), (Wbi, 'f')])


def tc_a2(h2, degp, Wd, bd, a, Wg, Wbi, n, blk):
    def body(h_r, d_r, wd_r, bd_r, a_r, wg_r, wbi_r, hsg_r, tb_r):
        invn = _inv_norm(d_r)
        hd = jnp.dot(h_r[...], wd_r[...],
                     preferred_element_type=jnp.float32) + bd_r[...]
        hd = jnp.where(hd >= 0, hd, a_r[0, 0] * hd)
        hsg_r[...] = jnp.dot(hd, wg_r[...],
                             preferred_element_type=jnp.float32) * invn
        tb_r[...] = jnp.dot(hd, wbi_r[...], preferred_element_type=jnp.float32)

    return _row_call(
        body, n, blk,
        [(_sd((n, D)), 'r'), (_sd((n, D)), 'r')],
        [(h2, 'r'), (degp, 'd'), (Wd, 'f'), (bd, 'f'), (a, 'f'),
         (Wg, 'f'), (Wbi, 'f')])


def tc_score(Q, hsg, tb, degp, bg, bbi, n, blk):
    def body(q_r, hsg_r, tb_r, d_r, bg_r, bbi_r, o_r):
        invn = _inv_norm(d_r)
        hx = invn * (q_r[0] + q_r[1] + hsg_r[...]) + bg_r[...]
        s = jnp.sum(tb_r[...] * hx, axis=1, keepdims=True) + bbi_r[0, 0]
        o_r[...] = jax.nn.sigmoid(s)

    return _row_call(
        body, n, blk, [(_sd((n, 1)), 'r')],
        [(Q, 'p'), (hsg, 'r'), (tb, 'r'), (degp, 'd'), (bg, 'f'),
         (bbi, 'f')])[0]


def tc_epilogue_mm(P, hs, degp, b, W, n, blk):
    def body(p_r, hs_r, d_r, b_r, w_r, o_r):
        invn = _inv_norm(d_r)
        hm = jax.nn.relu(invn * (p_r[0] + p_r[1] + hs_r[...]) + b_r[...])
        o_r[...] = jnp.dot(hm, w_r[...], preferred_element_type=jnp.float32)

    return _row_call(body, n, blk, [(_sd((n, D)), 'r')],
                     [(P, 'p'), (hs, 'r'), (degp, 'd'), (b, 'f'), (W, 'f')])[0]


def tc_gs_prep(mapping, degp, n, blk):
    def body(m_r, d_r, g_r, s_r):
        invn = _inv_norm(d_r)
        m = m_r[...]
        g_r[...] = jnp.maximum(m, 0)
        s_r[...] = jnp.where(m >= 0, invn, 0.0)

    return _row_call(
        body, n, blk,
        [(_sd((n, 1), jnp.int32), 'r'), (_sd((n, 1)), 'r')],
        [(mapping, 'r'), (degp, 'd')])


def tc_scale(raw, scl, n, blk):
    def body(x_r, s_r, o_r):
        o_r[...] = x_r[...] * s_r[...]

    return _row_call(body, n, blk, [(_sd((n, D)), 'r')],
                     [(raw, 'r'), (scl, 'r')])[0]


def tc_cross(Rm, hsm, U, hsf, degp, bm, bu, W, n, blk):
    def body(rm_r, hsm_r, u_r, hsf_r, d_r, bm_r, bu_r, w_r, o_r):
        invn = _inv_norm(d_r)
        hm = jax.nn.relu(invn * (rm_r[0] + rm_r[1] + hsm_r[...]) + bm_r[...])
        hc = hm + invn * (u_r[0] + u_r[1] + hsf_r[...]) + bu_r[...]
        o_r[...] = jnp.dot(hc, w_r[...], preferred_element_type=jnp.float32)

    return _row_call(
        body, n, blk, [(_sd((n, D)), 'r')],
        [(Rm, 'p'), (hsm, 'r'), (U, 'p'), (hsf, 'r'), (degp, 'd'),
         (bm, 'f'), (bu, 'f'), (W, 'f')])[0]


def tc_cross_final(Rm, hsm, U, hsf, degp, bm, bu, h1, WeA, WeB, n, blk):
    def body(rm_r, hsm_r, u_r, hsf_r, d_r, bm_r, bu_r, h1_r, wa_r, wb_r, o_r):
        invn = _inv_norm(d_r)
        hm = jax.nn.relu(invn * (rm_r[0] + rm_r[1] + hsm_r[...]) + bm_r[...])
        hc = hm + invn * (u_r[0] + u_r[1] + hsf_r[...]) + bu_r[...]
        o_r[...] = (jnp.dot(h1_r[...], wa_r[...],
                            preferred_element_type=jnp.float32)
                    + jnp.dot(hc, wb_r[...],
                              preferred_element_type=jnp.float32)) * invn

    return _row_call(
        body, n, blk, [(_sd((n, D)), 'r')],
        [(Rm, 'p'), (hsm, 'r'), (U, 'p'), (hsf, 'r'), (degp, 'd'),
         (bm, 'f'), (bu, 'f'), (h1, 'r'), (WeA, 'f'), (WeB, 'f')])[0]


def tc_final(P, hs, degp, b, n, blk):
    def body(p_r, hs_r, d_r, b_r, o_r):
        invn = _inv_norm(d_r)
        o_r[...] = invn * (p_r[0] + p_r[1] + hs_r[...]) + b_r[...]

    return _row_call(body, n, blk, [(_sd((n, D)), 'r')],
                     [(P, 'p'), (hs, 'r'), (degp, 'd'), (b, 'f')])[0]


def kernel(x, params, edge_index):
    p = params
    src = edge_index[0]
    dst = edge_index[1]
    pad = EP - E
    srcc = jnp.concatenate(
        [src, jnp.arange(pad, dtype=jnp.int32) % 4096]).reshape(NW, CPT, ECH)
    dstc = jnp.concatenate(
        [dst, N + (jnp.arange(pad, dtype=jnp.int32) % 32)]).reshape(
        NW, CPT, ECH)

    gs = lambda v: v.reshape(v.shape[0] // GCH, 1, GCH)
    r2 = lambda v: v.reshape(v.shape[0], 1)
    rd = lambda dg: dg.reshape(2, dg.shape[1] * dg.shape[3], 1)
    b = {k_: p[k_].reshape(1, D) for k_ in
         ('b0', 'bd1', 'bg1', 'bd2', 'bg2', 'bm1', 'bm2', 'bm3',
          'bu1', 'bu2', 'be')}
    s = {k_: p[k_].reshape(1, 1) for k_ in ('a1', 'bbi1', 'a2', 'bbi2')}

    degp1 = rd(sc_deg(dstc, NPAD1))

    hs0 = tc_mm_scale(x, p['W0'], degp1, N, 1000)
    P0 = sc_edge_agg(hs0, srcc, dstc, N, NPAD1)
    h1, hsg1, tb1 = tc_a1(P0, hs0, degp1, b['b0'], p['Wd1'], b['bd1'],
                          s['a1'], p['Wg1'], p['Wbi1'], N, 1000)
    Q1 = sc_edge_agg(hsg1, srcc, dstc, N, NPAD1)
    probs1 = tc_score(Q1, hsg1, tb1, degp1, b['bg1'], s['bbi1'], N, 1000)
    idx1, sc1, map1 = sc_topk(probs1.reshape(N), N, K1)

    srcc2, dstc2, degp2 = sc_relabel(map1, srcc, dstc, N, K1, NPAD2)
    degp2 = rd(degp2)
    h2 = tc_scale(sc_gather(h1, gs(idx1), K1).reshape(K1, D),
                  r2(sc1), K1, 1000)

    hsg2, tb2 = tc_a2(h2, degp2, p['Wd2'], b['bd2'], s['a2'], p['Wg2'],
                      p['Wbi2'], K1, 1000)
    Q2 = sc_edge_agg(hsg2, srcc2, dstc2, K1, NPAD2)
    probs2 = tc_score(Q2, hsg2, tb2, degp2, b['bg2'], s['bbi2'], K1, 1000)
    idx2, sc2, map2 = sc_topk(probs2.reshape(K1), K1, K2)

    srcc3, dstc3, degp3 = sc_relabel(map2, srcc2, dstc2, K1, K2, NPAD3)
    degp3 = rd(degp3)
    h3 = tc_scale(sc_gather(h2, gs(idx2), K2).reshape(K2, D),
                  r2(sc2), K2, 800)

    hsm1 = tc_mm_scale(h1, p['Wm1'], degp1, N, 1000)
    R1 = sc_edge_agg(hsm1, srcc, dstc, N, NPAD1)
    hsm2 = tc_mm_scale(h2, p['Wm2'], degp2, K1, 1000)
    R2 = sc_edge_agg(hsm2, srcc2, dstc2, K1, NPAD2)
    hsm3 = tc_mm_scale(h3, p['Wm3'], degp3, K2, 800)
    R3 = sc_edge_agg(hsm3, srcc3, dstc3, K2, NPAD3)

    fw3 = tc_epilogue_mm(R3, hsm3, degp3, b['bm3'], p['Wu2'], K2, 800)
    gidx2, scl2 = tc_gs_prep(r2(map2[:K1]), degp2, K1, 1000)
    hsf2 = tc_scale(sc_gather(fw3, gs(gidx2.reshape(K1)), K1).reshape(K1, D),
                    scl2, K1, 1000)
    U2 = sc_edge_agg(hsf2, srcc2, dstc2, K1, NPAD2)

    fw2 = tc_cross(R2, hsm2, U2, hsf2, degp2, b['bm2'], b['bu2'],
                   p['Wu1'], K1, 1000)
    gidx1, scl1 = tc_gs_prep(r2(map1[:N]), degp1, N, 1000)
    hsf1 = tc_scale(sc_gather(fw2, gs(gidx1.reshape(N)), N).reshape(N, D),
                    scl1, N, 1000)
    U1 = sc_edge_agg(hsf1, srcc, dstc, N, NPAD1)

    hse = tc_cross_final(R1, hsm1, U1, hsf1, degp1, b['bm1'], b['bu1'],
                         h1, p['We'][:D], p['We'][D:], N, 1000)
    Pe = sc_edge_agg(hse, srcc, dstc, N, NPAD1)
    out = tc_final(Pe, hse, degp1, b['be'], N, 1000)
    return out


# speedup vs baseline: 14.0774x; 1.1328x over previous
import functools
import math

import jax
import jax.numpy as jnp
from jax import lax
from jax.experimental import pallas as pl
from jax.experimental.pallas import tpu as pltpu
from jax.experimental.pallas import tpu_sc as plsc

N = 10000
E = 320000
D = 128
K1 = int(math.ceil(N * 0.8))
K2 = int(math.ceil(K1 * 0.7))

NC, NS, L = 2, 16, 16
NW = NC * NS

ECH = 128
NCHUNK_RAW = -(-E // ECH)
CPT = -(-NCHUNK_RAW // NW)
NCHUNK = CPT * NW
EP = NCHUNK * ECH
IBLK = 16

NPAD1 = 10112
NPAD2 = 8064
NPAD3 = 5632

_CP = pltpu.CompilerParams(needs_layout_passes=False)

_mesh = lambda: plsc.VectorSubcoreMesh(core_axis_name="c", subcore_axis_name="s")

_IOTA = lambda: lax.iota(jnp.int32, L)


def _mo8(v):
    return pl.multiple_of(v, 8)


def sc_edge_agg(hs, srcc, dstc, n, npad):
    zrows = npad // NS
    nfull, rem = divmod(zrows, ECH)
    nblk = -(-CPT // IBLK)

    @functools.partial(
        pl.kernel, mesh=_mesh(), compiler_params=_CP,
        out_type=jax.ShapeDtypeStruct((NC, NS, zrows, D), jnp.float32),
        scratch_types=[
            pltpu.VMEM((2, IBLK, ECH), jnp.int32),
            pltpu.VMEM((2, IBLK, ECH), jnp.int32),
            pltpu.VMEM((ECH, D), jnp.float32),
            pltpu.VMEM((ECH, D), jnp.float32),
            pltpu.VMEM_SHARED((npad, D), jnp.float32),
            pltpu.SemaphoreType.DMA,
            pltpu.SemaphoreType.DMA,
            pltpu.SemaphoreType.DMA,
        ],
    )
    def k(hs_h, src_h, dst_h, out_h, src_v, dst_v, rows0, rows1, acc,
          s0, s1, si):
        cid = lax.axis_index("c")
        sid = lax.axis_index("s")
        wid = sid * NC + cid
        zbase = _mo8(sid * zrows)

        def zb(i, _):
            rows0[i // (D // L),
                  pl.ds((i % (D // L)) * L, L)] = jnp.zeros((L,), jnp.float32)
            return 0

        lax.fori_loop(0, ECH * (D // L), zb, 0)
        for c in range(nfull):
            pltpu.sync_copy(rows0, acc.at[pl.ds(zbase + c * ECH, ECH)])
        if rem:
            pltpu.sync_copy(rows0.at[pl.ds(0, rem)],
                            acc.at[pl.ds(zbase + nfull * ECH, rem)])
        plsc.subcore_barrier()

        bufs = (rows0, rows1)
        sems = (s0, s1)
        pltpu.sync_copy(src_h.at[wid, pl.ds(0, IBLK)], src_v.at[0])
        pltpu.sync_copy(dst_h.at[wid, pl.ds(0, IBLK)], dst_v.at[0])
        gdescs = [None, None]
        gdescs[0] = pltpu.async_copy(hs_h.at[src_v.at[0, 0]], rows0, s0)
        for blk in range(nblk):
            bp = blk % 2
            nbp = (blk + 1) % 2
            bsz = min(IBLK, CPT - blk * IBLK)
            idescs = None
            for j in range(bsz):
                gc = blk * IBLK + j
                p = gc % 2
                q = (gc + 1) % 2
                gdescs[p].wait()
                if j == 0 and blk + 1 < nblk:
                    nbsz = min(IBLK, CPT - (blk + 1) * IBLK)
                    idescs = (
                        pltpu.async_copy(
                            src_h.at[wid, pl.ds((blk + 1) * IBLK, nbsz)],
                            src_v.at[nbp, pl.ds(0, nbsz)], si),
                        pltpu.async_copy(
                            dst_h.at[wid, pl.ds((blk + 1) * IBLK, nbsz)],
                            dst_v.at[nbp, pl.ds(0, nbsz)], si),
                    )
                if j + 1 < bsz:
                    gdescs[q] = pltpu.async_copy(
                        hs_h.at[src_v.at[bp, j + 1]], bufs[q], sems[q])
                elif idescs is not None:
                    idescs[0].wait()
                    idescs[1].wait()
                    gdescs[q] = pltpu.async_copy(
                        hs_h.at[src_v.at[nbp, 0]], bufs[q], sems[q])
                pltpu.sync_copy(bufs[p], acc.at[dst_v.at[bp, j]], add=True)
        plsc.subcore_barrier()

        pltpu.sync_copy(acc.at[pl.ds(zbase, zrows)], out_h.at[cid, sid])

    return k(hs, srcc, dstc).reshape(NC, npad, D)


def _deg_combine(dcount, sdeg, tmp, accv, deg_h, cid, sid, npad):
    spt = npad // NS
    pltpu.sync_copy(dcount, sdeg.at[sid])
    plsc.subcore_barrier()

    def zb(i, _):
        accv[0, pl.ds(i * L, L)] = jnp.zeros((L,), jnp.float32)
        return 0

    lax.fori_loop(0, spt // L, zb, 0)
    for r in range(NS):
        pltpu.sync_copy(sdeg.at[r, sid], tmp)

        def addb(i, _):
            accv[0, pl.ds(i * L, L)] = (accv[0, pl.ds(i * L, L)]
                                        + tmp[0, pl.ds(i * L, L)])
            return 0

        lax.fori_loop(0, spt // L, addb, 0)
    pltpu.sync_copy(accv, deg_h.at[cid, sid])


def _zero_dcount(dcount, spt):
    def zb(i, _):
        dcount[i // (spt // L), 0,
               pl.ds((i % (spt // L)) * L, L)] = jnp.zeros((L,), jnp.float32)
        return 0

    lax.fori_loop(0, NS * (spt // L), zb, 0)


def _count_into(dcount, idx, spt):
    cnt, last = plsc.scan_count(idx)
    hi = idx // spt
    lo = idx - hi * spt
    zero = jnp.zeros((L,), jnp.int32)
    plsc.addupdate_scatter(dcount, [hi, zero, lo], cnt.astype(jnp.float32),
                           mask=last)


def sc_deg(dstc, npad):
    spt = npad // NS

    @functools.partial(
        pl.kernel, mesh=_mesh(), compiler_params=_CP,
        out_type=jax.ShapeDtypeStruct((NC, NS, 1, spt), jnp.float32),
        scratch_types=[
            pltpu.VMEM((CPT, ECH), jnp.int32),
            pltpu.VMEM((NS, 1, spt), jnp.float32),
            pltpu.VMEM((1, spt), jnp.float32),
            pltpu.VMEM((1, spt), jnp.float32),
            pltpu.VMEM_SHARED((NS, NS, 1, spt), jnp.float32),
            pltpu.SemaphoreType.DMA,
        ],
    )
    def k(dst_h, deg_h, dst_v, dcount, tmp, accv, sdeg, sem):
        cid = lax.axis_index("c")
        sid = lax.axis_index("s")
        wid = sid * NC + cid
        _zero_dcount(dcount, spt)
        pltpu.sync_copy(dst_h.at[wid], dst_v)

        def body(c, _):
            for j in range(ECH // L):
                _count_into(dcount, dst_v[c, pl.ds(j * L, L)], spt)
            return 0

        lax.fori_loop(0, CPT, body, 0)
        _deg_combine(dcount, sdeg, tmp, accv, deg_h, cid, sid, npad)

    return k(dstc)


def sc_relabel(mapping, srcc, dstc, n_old, n_new, npad_new):
    spt = npad_new // NS

    @functools.partial(
        pl.kernel, mesh=_mesh(), compiler_params=_CP,
        out_type=[
            jax.ShapeDtypeStruct((NW, CPT, ECH), jnp.int32),
            jax.ShapeDtypeStruct((NW, CPT, ECH), jnp.int32),
            jax.ShapeDtypeStruct((NC, NS, 1, spt), jnp.float32),
        ],
        scratch_types=[
            pltpu.VMEM((n_old + 2 * L,), jnp.int32),
            pltpu.VMEM((CPT, ECH), jnp.int32),
            pltpu.VMEM((CPT, ECH), jnp.int32),
            pltpu.VMEM((CPT, ECH), jnp.int32),
            pltpu.VMEM((CPT, ECH), jnp.int32),
            pltpu.VMEM((NS, 1, spt), jnp.float32),
            pltpu.VMEM((1, spt), jnp.float32),
            pltpu.VMEM((1, spt), jnp.float32),
            pltpu.VMEM_SHARED((NS, NS, 1, spt), jnp.float32),
            pltpu.SemaphoreType.DMA,
        ],
    )
    def k(map_h, src_h, dst_h, src2_h, dst2_h, deg_h,
          map_v, src_v, dst_v, src2_v, dst2_v, dcount, tmp, accv, sdeg, sem):
        cid = lax.axis_index("c")
        sid = lax.axis_index("s")
        wid = sid * NC + cid
        _zero_dcount(dcount, spt)
        pltpu.sync_copy(map_h, map_v)
        pltpu.sync_copy(src_h.at[wid], src_v)
        pltpu.sync_copy(dst_h.at[wid], dst_v)

        zero = jnp.zeros((L,), jnp.int32)
        trash = jnp.full((L,), n_new, jnp.int32)
        m31 = jnp.full((L,), 31, jnp.int32)
        m4095 = jnp.full((L,), 4095, jnp.int32)

        def body(c, _):
            for j in range(ECH // L):
                s = src_v[c, pl.ds(j * L, L)]
                d = dst_v[c, pl.ds(j * L, L)]
                ns = plsc.load_gather(map_v, [s])
                nd = plsc.load_gather(map_v, [d])
                valid = (ns >= 0) & (nd >= 0)
                d2 = jnp.where(valid, nd, trash + (d & m31))
                src2_v[c, pl.ds(j * L, L)] = jnp.where(valid, ns, d & m4095)
                dst2_v[c, pl.ds(j * L, L)] = d2
                _count_into(dcount, d2, spt)
            return 0

        lax.fori_loop(0, CPT, body, 0)
        pltpu.sync_copy(src2_v, src2_h.at[wid])
        pltpu.sync_copy(dst2_v, dst2_h.at[wid])
        _deg_combine(dcount, sdeg, tmp, accv, deg_h, cid, sid, npad_new)

    return k(mapping, srcc, dstc)


def sc_topk(probs, n, kk):
    nv = n // L
    UNR = 4
    nvu, nvrem = divmod(nv, UNR)

    @functools.partial(
        pl.kernel, mesh=_mesh(), compiler_params=_CP,
        out_type=[
            jax.ShapeDtypeStruct((kk,), jnp.int32),
            jax.ShapeDtypeStruct((kk,), jnp.float32),
            jax.ShapeDtypeStruct((n + 2 * L,), jnp.int32),
        ],
        scratch_types=[
            pltpu.VMEM((n,), jnp.float32),
            pltpu.VMEM((kk,), jnp.int32),
            pltpu.VMEM((kk,), jnp.float32),
            pltpu.VMEM((n + 2 * L,), jnp.int32),
            pltpu.SemaphoreType.DMA,
        ],
    )
    def k(p_h, sel_h, sv_h, map_h, pv, sel_v, sv_v, map_v, sem):
        cid = lax.axis_index("c")
        sid = lax.axis_index("s")

        @pl.when((cid == 0) & (sid == 0))
        def _():
            pltpu.sync_copy(p_h, pv)
            kvec = jnp.full((L,), kk, jnp.int32)
            one = jnp.ones((L,), jnp.int32)
            zero = jnp.zeros((L,), jnp.int32)

            def lane_count(pred):
                def cbody(j, accs):
                    a0, a1 = accs
                    for u in range(UNR):
                        uv = plsc.bitcast(
                            pv[pl.ds((j * UNR + u) * L, L)], jnp.int32)
                        x = jnp.where(pred(uv), one, zero)
                        if u % 2 == 0:
                            a0 = a0 + x
                        else:
                            a1 = a1 + x
                    return a0, a1

                a0, a1 = lax.fori_loop(0, nvu, cbody, (zero, zero))
                for u in range(nvrem):
                    uv = plsc.bitcast(
                        pv[pl.ds((nvu * UNR + u) * L, L)], jnp.int32)
                    a0 = a0 + jnp.where(pred(uv), one, zero)
                tot = jnp.sum(a0 + a1)
                return jnp.broadcast_to(tot, (L,))

            def sround(i, thr):
                b = jnp.broadcast_to(30 - i, (L,)).astype(jnp.int32)
                cand = thr | (one << b)
                cnt = lane_count(lambda uv: uv >= cand)
                return jnp.where(cnt >= kvec, cand, thr)

            thr = lax.fori_loop(0, 31, sround, zero)
            g_cnt = lane_count(lambda uv: uv > thr)
            need_eq = kvec - g_cnt

            def mbody(i, _):
                map_v[pl.ds(i * L, L)] = jnp.full((L,), -1, jnp.int32)
                return 0

            lax.fori_loop(0, nv + 2, mbody, 0)
            iota = _IOTA()

            def sbody(j, carry):
                off, eqs = carry
                v = pv[pl.ds(j * L, L)]
                u = plsc.bitcast(v, jnp.int32)
                m_gt = u > thr
                m_eq = u == thr
                eqpos = plsc.cumsum(jnp.where(m_eq, one, zero))
                m = m_gt | (m_eq & ((eqs + eqpos) <= need_eq))
                rank = off + plsc.cumsum(jnp.where(m, one, zero)) - one
                idxv = jnp.full((L,), j * L, jnp.int32) + iota
                plsc.store_scatter(sel_v, [rank], idxv, mask=m)
                plsc.store_scatter(sv_v, [rank], v, mask=m)
                plsc.store_scatter(map_v, [idxv], rank, mask=m)
                off = off + plsc.all_reduce_population_count(m)
                eqs = eqs + plsc.all_reduce_population_count(m_eq)
                return off, eqs

            lax.fori_loop(0, nv, sbody, (zero, zero))
            pltpu.sync_copy(sel_v, sel_h)
            pltpu.sync_copy(sv_v, sv_h)
            pltpu.sync_copy(map_v, map_h)

    return k(probs)


GCH = 80


def sc_gather(tbl, gidx, nout):
    nch = nout // GCH
    iters = -(-nch // NW)

    @functools.partial(
        pl.kernel, mesh=_mesh(), compiler_params=_CP,
        out_type=jax.ShapeDtypeStruct((nch, GCH, D), jnp.float32),
        scratch_types=[
            pltpu.VMEM((1, GCH), jnp.int32),
            pltpu.VMEM((GCH, D), jnp.float32),
            pltpu.SemaphoreType.DMA,
        ],
    )
    def k(tbl_h, gidx_h, out_h, idx_v, rows_v, sem):
        cid = lax.axis_index("c")
        sid = lax.axis_index("s")
        wid = sid * NC + cid
        for it in range(iters):
            c = it * NW + wid

            @pl.when(c < nch)
            def _():
                pltpu.sync_copy(gidx_h.at[c], idx_v)
                pltpu.async_copy(tbl_h.at[idx_v.at[0]], rows_v, sem).wait()
                pltpu.sync_copy(rows_v, out_h.at[c])

    return k(tbl, gidx)


def _inv_norm(dref):
    return lax.rsqrt(1.0 + dref[0] + dref[1])


def _row_call(body, n, blk, outs, ins):
    grid = (n // blk,)

    def spec(a, kind):
        if kind == 'r':
            return pl.BlockSpec((blk, a.shape[1]), lambda i: (i, 0))
        if kind in ('d', 'p'):
            return pl.BlockSpec((2, blk, a.shape[2]), lambda i: (0, i, 0))
        return pl.BlockSpec(a.shape, lambda i: tuple(0 for _ in a.shape))

    return pl.pallas_call(
        body,
        grid=grid,
        in_specs=[spec(a, kd) for a, kd in ins],
        out_specs=[spec(a, kd) for a, kd in outs],
        out_shape=[jax.ShapeDtypeStruct(a.shape, a.dtype) for a, _ in outs],
    )(*[a for a, _ in ins])


def _sd(shape, dtype=jnp.float32):
    return jax.ShapeDtypeStruct(shape, dtype)


def tc_mm_scale(x, W, degp, n, blk):
    def body(x_r, d_r, w_r, o_r):
        invn = _inv_norm(d_r)
        o_r[...] = jnp.dot(x_r[...], w_r[...],
                           preferred_element_type=jnp.float32) * invn

    return _row_call(body, n, blk, [(_sd((n, D)), 'r')],
                     [(x, 'r'), (degp, 'd'), (W, 'f')])[0]


def tc_a1(P, hs0, degp, b0, Wd, bd, a, Wg, Wbi, n, blk):
    def body(p_r, hs_r, d_r, b0_r, wd_r, bd_r, a_r, wg_r, wbi_r,
             h1_r, hsg_r, tb_r):
        invn = _inv_norm(d_r)
        h1 = jax.nn.relu(invn * (p_r[0] + p_r[1] + hs_r[...]) + b0_r[...])
        h1_r[...] = h1
        hd = jnp.dot(h1, wd_r[...], preferred_element_type=jnp.float32) \
            + bd_r[...]
        hd = jnp.where(hd >= 0, hd, a_r[0, 0] * hd)
        hsg_r[...] = jnp.dot(hd, wg_r[...],
                             preferred_element_type=jnp.float32) * invn
        tb_r[...] = jnp.dot(hd, wbi_r[...], preferred_element_type=jnp.float32)

    return _row_call(
        body, n, blk,
        [(_sd((n, D)), 'r'), (_sd((n, D)), 'r'), (_sd((n, D)), 'r')],
        [(P, 'p'), (hs0, 'r'), (degp, 'd'), (b0, 'f'), (Wd, 'f'),
         (bd, 'f'), (a, 'f'), (Wg, 'f'), (Wbi, 'f')])


def tc_a2(h2, degp, Wd, bd, a, Wg, Wbi, n, blk):
    def body(h_r, d_r, wd_r, bd_r, a_r, wg_r, wbi_r, hsg_r, tb_r):
        invn = _inv_norm(d_r)
        hd = jnp.dot(h_r[...], wd_r[...],
                     preferred_element_type=jnp.float32) + bd_r[...]
        hd = jnp.where(hd >= 0, hd, a_r[0, 0] * hd)
        hsg_r[...] = jnp.dot(hd, wg_r[...],
                             preferred_element_type=jnp.float32) * invn
        tb_r[...] = jnp.dot(hd, wbi_r[...], preferred_element_type=jnp.float32)

    return _row_call(
        body, n, blk,
        [(_sd((n, D)), 'r'), (_sd((n, D)), 'r')],
        [(h2, 'r'), (degp, 'd'), (Wd, 'f'), (bd, 'f'), (a, 'f'),
         (Wg, 'f'), (Wbi, 'f')])


def tc_score(Q, hsg, tb, degp, bg, bbi, n, blk):
    def body(q_r, hsg_r, tb_r, d_r, bg_r, bbi_r, o_r):
        invn = _inv_norm(d_r)
        hx = invn * (q_r[0] + q_r[1] + hsg_r[...]) + bg_r[...]
        s = jnp.sum(tb_r[...] * hx, axis=1, keepdims=True) + bbi_r[0, 0]
        o_r[...] = jax.nn.sigmoid(s)

    return _row_call(
        body, n, blk, [(_sd((n, 1)), 'r')],
        [(Q, 'p'), (hsg, 'r'), (tb, 'r'), (degp, 'd'), (bg, 'f'),
         (bbi, 'f')])[0]


def tc_epilogue_mm(P, hs, degp, b, W, n, blk):
    def body(p_r, hs_r, d_r, b_r, w_r, o_r):
        invn = _inv_norm(d_r)
        hm = jax.nn.relu(invn * (p_r[0] + p_r[1] + hs_r[...]) + b_r[...])
        o_r[...] = jnp.dot(hm, w_r[...], preferred_element_type=jnp.float32)

    return _row_call(body, n, blk, [(_sd((n, D)), 'r')],
                     [(P, 'p'), (hs, 'r'), (degp, 'd'), (b, 'f'), (W, 'f')])[0]


def tc_gs_prep(mapping, degp, n, blk):
    def body(m_r, d_r, g_r, s_r):
        invn = _inv_norm(d_r)
        m = m_r[...]
        rows = (lax.broadcasted_iota(jnp.int32, (blk, 1), 0)
                + pl.program_id(0) * blk) & 4095
        g_r[...] = jnp.where(m >= 0, m, rows)
        s_r[...] = jnp.where(m >= 0, invn, 0.0)

    return _row_call(
        body, n, blk,
        [(_sd((n, 1), jnp.int32), 'r'), (_sd((n, 1)), 'r')],
        [(mapping, 'r'), (degp, 'd')])


def tc_scale(raw, scl, n, blk):
    def body(x_r, s_r, o_r):
        o_r[...] = x_r[...] * s_r[...]

    return _row_call(body, n, blk, [(_sd((n, D)), 'r')],
                     [(raw, 'r'), (scl, 'r')])[0]


def tc_cross(Rm, hsm, U, hsf, degp, bm, bu, W, n, blk):
    def body(rm_r, hsm_r, u_r, hsf_r, d_r, bm_r, bu_r, w_r, o_r):
        invn = _inv_norm(d_r)
        hm = jax.nn.relu(invn * (rm_r[0] + rm_r[1] + hsm_r[...]) + bm_r[...])
        hc = hm + invn * (u_r[0] + u_r[1] + hsf_r[...]) + bu_r[...]
        o_r[...] = jnp.dot(hc, w_r[...], preferred_element_type=jnp.float32)

    return _row_call(
        body, n, blk, [(_sd((n, D)), 'r')],
        [(Rm, 'p'), (hsm, 'r'), (U, 'p'), (hsf, 'r'), (degp, 'd'),
         (bm, 'f'), (bu, 'f'), (W, 'f')])[0]


def tc_cross_final(Rm, hsm, U, hsf, degp, bm, bu, h1, WeA, WeB, n, blk):
    def body(rm_r, hsm_r, u_r, hsf_r, d_r, bm_r, bu_r, h1_r, wa_r, wb_r, o_r):
        invn = _inv_norm(d_r)
        hm = jax.nn.relu(invn * (rm_r[0] + rm_r[1] + hsm_r[...]) + bm_r[...])
        hc = hm + invn * (u_r[0] + u_r[1] + hsf_r[...]) + bu_r[...]
        o_r[...] = (jnp.dot(h1_r[...], wa_r[...],
                            preferred_element_type=jnp.float32)
                    + jnp.dot(hc, wb_r[...],
                              preferred_element_type=jnp.float32)) * invn

    return _row_call(
        body, n, blk, [(_sd((n, D)), 'r')],
        [(Rm, 'p'), (hsm, 'r'), (U, 'p'), (hsf, 'r'), (degp, 'd'),
         (bm, 'f'), (bu, 'f'), (h1, 'r'), (WeA, 'f'), (WeB, 'f')])[0]


def tc_final(P, hs, degp, b, n, blk):
    def body(p_r, hs_r, d_r, b_r, o_r):
        invn = _inv_norm(d_r)
        o_r[...] = invn * (p_r[0] + p_r[1] + hs_r[...]) + b_r[...]

    return _row_call(body, n, blk, [(_sd((n, D)), 'r')],
                     [(P, 'p'), (hs, 'r'), (degp, 'd'), (b, 'f')])[0]


def kernel(x, params, edge_index):
    p = params
    src = edge_index[0]
    dst = edge_index[1]
    pad = EP - E
    srcc = jnp.concatenate(
        [src, jnp.arange(pad, dtype=jnp.int32) % 4096]).reshape(NW, CPT, ECH)
    dstc = jnp.concatenate(
        [dst, N + (jnp.arange(pad, dtype=jnp.int32) % 32)]).reshape(
        NW, CPT, ECH)

    gs = lambda v: v.reshape(v.shape[0] // GCH, 1, GCH)
    r2 = lambda v: v.reshape(v.shape[0], 1)
    rd = lambda dg: dg.reshape(2, dg.shape[1] * dg.shape[3], 1)
    b = {k_: p[k_].reshape(1, D) for k_ in
         ('b0', 'bd1', 'bg1', 'bd2', 'bg2', 'bm1', 'bm2', 'bm3',
          'bu1', 'bu2', 'be')}
    s = {k_: p[k_].reshape(1, 1) for k_ in ('a1', 'bbi1', 'a2', 'bbi2')}

    degp1 = rd(sc_deg(dstc, NPAD1))

    hs0 = tc_mm_scale(x, p['W0'], degp1, N, 1000)
    P0 = sc_edge_agg(hs0, srcc, dstc, N, NPAD1)
    h1, hsg1, tb1 = tc_a1(P0, hs0, degp1, b['b0'], p['Wd1'], b['bd1'],
                          s['a1'], p['Wg1'], p['Wbi1'], N, 1000)
    Q1 = sc_edge_agg(hsg1, srcc, dstc, N, NPAD1)
    probs1 = tc_score(Q1, hsg1, tb1, degp1, b['bg1'], s['bbi1'], N, 1000)
    idx1, sc1, map1 = sc_topk(probs1.reshape(N), N, K1)

    srcc2, dstc2, degp2 = sc_relabel(map1, srcc, dstc, N, K1, NPAD2)
    degp2 = rd(degp2)
    h2 = tc_scale(sc_gather(h1, gs(idx1), K1).reshape(K1, D),
                  r2(sc1), K1, 1000)

    hsg2, tb2 = tc_a2(h2, degp2, p['Wd2'], b['bd2'], s['a2'], p['Wg2'],
                      p['Wbi2'], K1, 1000)
    Q2 = sc_edge_agg(hsg2, srcc2, dstc2, K1, NPAD2)
    probs2 = tc_score(Q2, hsg2, tb2, degp2, b['bg2'], s['bbi2'], K1, 1000)
    idx2, sc2, map2 = sc_topk(probs2.reshape(K1), K1, K2)

    srcc3, dstc3, degp3 = sc_relabel(map2, srcc2, dstc2, K1, K2, NPAD3)
    degp3 = rd(degp3)
    h3 = tc_scale(sc_gather(h2, gs(idx2), K2).reshape(K2, D),
                  r2(sc2), K2, 800)

    hsm1 = tc_mm_scale(h1, p['Wm1'], degp1, N, 1000)
    R1 = sc_edge_agg(hsm1, srcc, dstc, N, NPAD1)
    hsm2 = tc_mm_scale(h2, p['Wm2'], degp2, K1, 1000)
    R2 = sc_edge_agg(hsm2, srcc2, dstc2, K1, NPAD2)
    hsm3 = tc_mm_scale(h3, p['Wm3'], degp3, K2, 800)
    R3 = sc_edge_agg(hsm3, srcc3, dstc3, K2, NPAD3)

    fw3 = tc_epilogue_mm(R3, hsm3, degp3, b['bm3'], p['Wu2'], K2, 800)
    gidx2, scl2 = tc_gs_prep(r2(map2[:K1]), degp2, K1, 1000)
    hsf2 = tc_scale(sc_gather(fw3, gs(gidx2.reshape(K1)), K1).reshape(K1, D),
                    scl2, K1, 1000)
    U2 = sc_edge_agg(hsf2, srcc2, dstc2, K1, NPAD2)

    fw2 = tc_cross(R2, hsm2, U2, hsf2, degp2, b['bm2'], b['bu2'],
                   p['Wu1'], K1, 1000)
    gidx1, scl1 = tc_gs_prep(r2(map1[:N]), degp1, N, 1000)
    hsf1 = tc_scale(sc_gather(fw2, gs(gidx1.reshape(N)), N).reshape(N, D),
                    scl1, N, 1000)
    U1 = sc_edge_agg(hsf1, srcc, dstc, N, NPAD1)

    hse = tc_cross_final(R1, hsm1, U1, hsf1, degp1, b['bm1'], b['bu1'],
                         h1, p['We'][:D], p['We'][D:], N, 1000)
    Pe = sc_edge_agg(hse, srcc, dstc, N, NPAD1)
    out = tc_final(Pe, hse, degp1, b['be'], N, 1000)
    return out


# speedup vs baseline: 14.6616x; 1.0415x over previous
import functools
import math

import jax
import jax.numpy as jnp
from jax import lax
from jax.experimental import pallas as pl
from jax.experimental.pallas import tpu as pltpu
from jax.experimental.pallas import tpu_sc as plsc

N = 10000
E = 320000
D = 128
K1 = int(math.ceil(N * 0.8))
K2 = int(math.ceil(K1 * 0.7))

NC, NS, L = 2, 16, 16
NW = NC * NS

ECH = 128
NCHUNK_RAW = -(-E // ECH)
CPT = -(-NCHUNK_RAW // NW)
NCHUNK = CPT * NW
EP = NCHUNK * ECH
IBLK = 16

NPAD1 = 10112
NPAD2 = 8064
NPAD3 = 5760

_CP = pltpu.CompilerParams(needs_layout_passes=False)

_mesh = lambda: plsc.VectorSubcoreMesh(core_axis_name="c", subcore_axis_name="s")

_IOTA = lambda: lax.iota(jnp.int32, L)


def _mo8(v):
    return pl.multiple_of(v, 8)


def sc_edge_agg(hs, srcc, dstc, n, npad):
    zrows = npad // NS
    nfull, rem = divmod(zrows, ECH)
    nblk = -(-CPT // IBLK)

    @functools.partial(
        pl.kernel, mesh=_mesh(), compiler_params=_CP,
        out_type=jax.ShapeDtypeStruct((NC, NS, zrows, D), jnp.float32),
        scratch_types=[
            pltpu.VMEM((2, IBLK, ECH), jnp.int32),
            pltpu.VMEM((2, IBLK, ECH), jnp.int32),
            pltpu.VMEM((ECH, D), jnp.float32),
            pltpu.VMEM((ECH, D), jnp.float32),
            pltpu.VMEM_SHARED((npad, D), jnp.float32),
            pltpu.SemaphoreType.DMA,
            pltpu.SemaphoreType.DMA,
            pltpu.SemaphoreType.DMA,
        ],
    )
    def k(hs_h, src_h, dst_h, out_h, src_v, dst_v, rows0, rows1, acc,
          s0, s1, si):
        cid = lax.axis_index("c")
        sid = lax.axis_index("s")
        wid = sid * NC + cid
        zbase = _mo8(sid * zrows)

        def zb(i, _):
            rows0[i // (D // L),
                  pl.ds((i % (D // L)) * L, L)] = jnp.zeros((L,), jnp.float32)
            return 0

        lax.fori_loop(0, ECH * (D // L), zb, 0)
        for c in range(nfull):
            pltpu.sync_copy(rows0, acc.at[pl.ds(zbase + c * ECH, ECH)])
        if rem:
            pltpu.sync_copy(rows0.at[pl.ds(0, rem)],
                            acc.at[pl.ds(zbase + nfull * ECH, rem)])
        plsc.subcore_barrier()

        bufs = (rows0, rows1)
        sems = (s0, s1)
        pltpu.sync_copy(src_h.at[wid, pl.ds(0, IBLK)], src_v.at[0])
        pltpu.sync_copy(dst_h.at[wid, pl.ds(0, IBLK)], dst_v.at[0])
        gdescs = [None, None]
        gdescs[0] = pltpu.async_copy(hs_h.at[src_v.at[0, 0]], rows0, s0)
        for blk in range(nblk):
            bp = blk % 2
            nbp = (blk + 1) % 2
            bsz = min(IBLK, CPT - blk * IBLK)
            idescs = None
            for j in range(bsz):
                gc = blk * IBLK + j
                p = gc % 2
                q = (gc + 1) % 2
                gdescs[p].wait()
                if j == 0 and blk + 1 < nblk:
                    nbsz = min(IBLK, CPT - (blk + 1) * IBLK)
                    idescs = (
                        pltpu.async_copy(
                            src_h.at[wid, pl.ds((blk + 1) * IBLK, nbsz)],
                            src_v.at[nbp, pl.ds(0, nbsz)], si),
                        pltpu.async_copy(
                            dst_h.at[wid, pl.ds((blk + 1) * IBLK, nbsz)],
                            dst_v.at[nbp, pl.ds(0, nbsz)], si),
                    )
                if j + 1 < bsz:
                    gdescs[q] = pltpu.async_copy(
                        hs_h.at[src_v.at[bp, j + 1]], bufs[q], sems[q])
                elif idescs is not None:
                    idescs[0].wait()
                    idescs[1].wait()
                    gdescs[q] = pltpu.async_copy(
                        hs_h.at[src_v.at[nbp, 0]], bufs[q], sems[q])
                pltpu.sync_copy(bufs[p], acc.at[dst_v.at[bp, j]], add=True)
        plsc.subcore_barrier()

        pltpu.sync_copy(acc.at[pl.ds(zbase, zrows)], out_h.at[cid, sid])

    return k(hs, srcc, dstc).reshape(NC, npad, D)


def _deg_combine(dcount, sdeg, tmp, accv, deg_h, cid, sid, npad):
    spt = npad // NS
    pltpu.sync_copy(dcount, sdeg.at[sid])
    plsc.subcore_barrier()

    def zb(i, _):
        accv[0, pl.ds(i * L, L)] = jnp.zeros((L,), jnp.float32)
        return 0

    lax.fori_loop(0, spt // L, zb, 0)
    for r in range(NS):
        pltpu.sync_copy(sdeg.at[r, sid], tmp)

        def addb(i, _):
            accv[0, pl.ds(i * L, L)] = (accv[0, pl.ds(i * L, L)]
                                        + tmp[0, pl.ds(i * L, L)])
            return 0

        lax.fori_loop(0, spt // L, addb, 0)
    pltpu.sync_copy(accv, deg_h.at[cid, sid])


def _zero_dcount(dcount, spt):
    def zb(i, _):
        dcount[i // (spt // L), 0,
               pl.ds((i % (spt // L)) * L, L)] = jnp.zeros((L,), jnp.float32)
        return 0

    lax.fori_loop(0, NS * (spt // L), zb, 0)


def _count_into(dcount, idx, spt):
    cnt, last = plsc.scan_count(idx)
    hi = idx // spt
    lo = idx - hi * spt
    zero = jnp.zeros((L,), jnp.int32)
    plsc.addupdate_scatter(dcount, [hi, zero, lo], cnt.astype(jnp.float32),
                           mask=last)


def sc_deg(dstc, npad):
    spt = npad // NS

    @functools.partial(
        pl.kernel, mesh=_mesh(), compiler_params=_CP,
        out_type=jax.ShapeDtypeStruct((NC, NS, 1, spt), jnp.float32),
        scratch_types=[
            pltpu.VMEM((CPT, ECH), jnp.int32),
            pltpu.VMEM((NS, 1, spt), jnp.float32),
            pltpu.VMEM((1, spt), jnp.float32),
            pltpu.VMEM((1, spt), jnp.float32),
            pltpu.VMEM_SHARED((NS, NS, 1, spt), jnp.float32),
            pltpu.SemaphoreType.DMA,
        ],
    )
    def k(dst_h, deg_h, dst_v, dcount, tmp, accv, sdeg, sem):
        cid = lax.axis_index("c")
        sid = lax.axis_index("s")
        wid = sid * NC + cid
        _zero_dcount(dcount, spt)
        pltpu.sync_copy(dst_h.at[wid], dst_v)

        def body(c, _):
            for j in range(ECH // L):
                _count_into(dcount, dst_v[c, pl.ds(j * L, L)], spt)
            return 0

        lax.fori_loop(0, CPT, body, 0)
        _deg_combine(dcount, sdeg, tmp, accv, deg_h, cid, sid, npad)

    return k(dstc)


def sc_relabel(mapping, srcc, dstc, n_old, n_new, npad_new, tmask):
    spt = npad_new // NS

    @functools.partial(
        pl.kernel, mesh=_mesh(), compiler_params=_CP,
        out_type=[
            jax.ShapeDtypeStruct((NW, CPT, ECH), jnp.int32),
            jax.ShapeDtypeStruct((NW, CPT, ECH), jnp.int32),
            jax.ShapeDtypeStruct((NC, NS, 1, spt), jnp.float32),
        ],
        scratch_types=[
            pltpu.VMEM((n_old + 4 * L,), jnp.int32),
            pltpu.VMEM((CPT, ECH), jnp.int32),
            pltpu.VMEM((CPT, ECH), jnp.int32),
            pltpu.VMEM((CPT, ECH), jnp.int32),
            pltpu.VMEM((CPT, ECH), jnp.int32),
            pltpu.VMEM((NS, 1, spt), jnp.float32),
            pltpu.VMEM((1, spt), jnp.float32),
            pltpu.VMEM((1, spt), jnp.float32),
            pltpu.VMEM_SHARED((NS, NS, 1, spt), jnp.float32),
            pltpu.SemaphoreType.DMA,
        ],
    )
    def k(map_h, src_h, dst_h, src2_h, dst2_h, deg_h,
          map_v, src_v, dst_v, src2_v, dst2_v, dcount, tmp, accv, sdeg, sem):
        cid = lax.axis_index("c")
        sid = lax.axis_index("s")
        wid = sid * NC + cid
        _zero_dcount(dcount, spt)
        pltpu.sync_copy(map_h, map_v)
        pltpu.sync_copy(src_h.at[wid], src_v)
        pltpu.sync_copy(dst_h.at[wid], dst_v)

        zero = jnp.zeros((L,), jnp.int32)
        trash = jnp.full((L,), n_new, jnp.int32)
        m31 = jnp.full((L,), tmask, jnp.int32)
        m4095 = jnp.full((L,), 4095, jnp.int32)

        def body(c, _):
            for j in range(ECH // L):
                s = src_v[c, pl.ds(j * L, L)]
                d = dst_v[c, pl.ds(j * L, L)]
                ns = plsc.load_gather(map_v, [s])
                nd = plsc.load_gather(map_v, [d])
                valid = (ns >= 0) & (nd >= 0)
                d2 = jnp.where(valid, nd, trash + (d & m31))
                src2_v[c, pl.ds(j * L, L)] = jnp.where(valid, ns, d & m4095)
                dst2_v[c, pl.ds(j * L, L)] = d2
                _count_into(dcount, d2, spt)
            return 0

        lax.fori_loop(0, CPT, body, 0)
        pltpu.sync_copy(src2_v, src2_h.at[wid])
        pltpu.sync_copy(dst2_v, dst2_h.at[wid])
        _deg_combine(dcount, sdeg, tmp, accv, deg_h, cid, sid, npad_new)

    return k(mapping, srcc, dstc)


def sc_topk(probs, n, kk):
    nv = n // L
    UNR = 4
    nvu, nvrem = divmod(nv, UNR)

    @functools.partial(
        pl.kernel, mesh=_mesh(), compiler_params=_CP,
        out_type=[
            jax.ShapeDtypeStruct((kk,), jnp.int32),
            jax.ShapeDtypeStruct((kk,), jnp.float32),
            jax.ShapeDtypeStruct((n + 4 * L,), jnp.int32),
        ],
        scratch_types=[
            pltpu.VMEM((n,), jnp.float32),
            pltpu.VMEM((kk,), jnp.int32),
            pltpu.VMEM((kk,), jnp.float32),
            pltpu.VMEM((n + 4 * L,), jnp.int32),
            pltpu.SemaphoreType.DMA,
        ],
    )
    def k(p_h, sel_h, sv_h, map_h, pv, sel_v, sv_v, map_v, sem):
        cid = lax.axis_index("c")
        sid = lax.axis_index("s")

        @pl.when((cid == 0) & (sid == 0))
        def _():
            pltpu.sync_copy(p_h, pv)
            kvec = jnp.full((L,), kk, jnp.int32)
            one = jnp.ones((L,), jnp.int32)
            zero = jnp.zeros((L,), jnp.int32)

            def lane_count(pred):
                def cbody(j, accs):
                    a0, a1 = accs
                    for u in range(UNR):
                        uv = plsc.bitcast(
                            pv[pl.ds((j * UNR + u) * L, L)], jnp.int32)
                        x = jnp.where(pred(uv), one, zero)
                        if u % 2 == 0:
                            a0 = a0 + x
                        else:
                            a1 = a1 + x
                    return a0, a1

                a0, a1 = lax.fori_loop(0, nvu, cbody, (zero, zero))
                for u in range(nvrem):
                    uv = plsc.bitcast(
                        pv[pl.ds((nvu * UNR + u) * L, L)], jnp.int32)
                    a0 = a0 + jnp.where(pred(uv), one, zero)
                tot = jnp.sum(a0 + a1)
                return jnp.broadcast_to(tot, (L,))

            def sround(i, thr):
                b = jnp.broadcast_to(30 - i, (L,)).astype(jnp.int32)
                cand = thr | (one << b)
                cnt = lane_count(lambda uv: uv >= cand)
                return jnp.where(cnt >= kvec, cand, thr)

            thr = lax.fori_loop(0, 31, sround, zero)
            g_cnt = lane_count(lambda uv: uv > thr)
            need_eq = kvec - g_cnt

            def mbody(i, _):
                map_v[pl.ds(i * L, L)] = jnp.full((L,), -1, jnp.int32)
                return 0

            lax.fori_loop(0, nv + 4, mbody, 0)
            iota = _IOTA()

            def sbody(j, carry):
                off, eqs = carry
                v = pv[pl.ds(j * L, L)]
                u = plsc.bitcast(v, jnp.int32)
                m_gt = u > thr
                m_eq = u == thr
                eqpos = plsc.cumsum(jnp.where(m_eq, one, zero))
                m = m_gt | (m_eq & ((eqs + eqpos) <= need_eq))
                rank = off + plsc.cumsum(jnp.where(m, one, zero)) - one
                idxv = jnp.full((L,), j * L, jnp.int32) + iota
                plsc.store_scatter(sel_v, [rank], idxv, mask=m)
                plsc.store_scatter(sv_v, [rank], v, mask=m)
                plsc.store_scatter(map_v, [idxv], rank, mask=m)
                off = off + plsc.all_reduce_population_count(m)
                eqs = eqs + plsc.all_reduce_population_count(m_eq)
                return off, eqs

            lax.fori_loop(0, nv, sbody, (zero, zero))
            pltpu.sync_copy(sel_v, sel_h)
            pltpu.sync_copy(sv_v, sv_h)
            pltpu.sync_copy(map_v, map_h)

    return k(probs)


GCH = 80


def sc_gather(tbl, gidx, nout):
    nch = nout // GCH
    iters = -(-nch // NW)

    @functools.partial(
        pl.kernel, mesh=_mesh(), compiler_params=_CP,
        out_type=jax.ShapeDtypeStruct((nch, GCH, D), jnp.float32),
        scratch_types=[
            pltpu.VMEM((1, GCH), jnp.int32),
            pltpu.VMEM((GCH, D), jnp.float32),
            pltpu.SemaphoreType.DMA,
        ],
    )
    def k(tbl_h, gidx_h, out_h, idx_v, rows_v, sem):
        cid = lax.axis_index("c")
        sid = lax.axis_index("s")
        wid = sid * NC + cid
        for it in range(iters):
            c = it * NW + wid

            @pl.when(c < nch)
            def _():
                pltpu.sync_copy(gidx_h.at[c], idx_v)
                pltpu.async_copy(tbl_h.at[idx_v.at[0]], rows_v, sem).wait()
                pltpu.sync_copy(rows_v, out_h.at[c])

    return k(tbl, gidx)


def _inv_norm(dref):
    return lax.rsqrt(1.0 + dref[0] + dref[1])


def _row_call(body, n, blk, outs, ins):
    grid = (n // blk,)

    def spec(a, kind):
        if kind == 'r':
            return pl.BlockSpec((blk, a.shape[1]), lambda i: (i, 0))
        if kind in ('d', 'p'):
            return pl.BlockSpec((2, blk, a.shape[2]), lambda i: (0, i, 0))
        return pl.BlockSpec(a.shape, lambda i: tuple(0 for _ in a.shape))

    return pl.pallas_call(
        body,
        grid=grid,
        in_specs=[spec(a, kd) for a, kd in ins],
        out_specs=[spec(a, kd) for a, kd in outs],
        out_shape=[jax.ShapeDtypeStruct(a.shape, a.dtype) for a, _ in outs],
    )(*[a for a, _ in ins])


def _sd(shape, dtype=jnp.float32):
    return jax.ShapeDtypeStruct(shape, dtype)


def tc_mm_scale(x, W, degp, n, blk):
    def body(x_r, d_r, w_r, o_r):
        invn = _inv_norm(d_r)
        o_r[...] = jnp.dot(x_r[...], w_r[...],
                           preferred_element_type=jnp.float32) * invn

    return _row_call(body, n, blk, [(_sd((n, D)), 'r')],
                     [(x, 'r'), (degp, 'd'), (W, 'f')])[0]


def tc_a1(P, hs0, degp, b0, Wd, bd, a, Wg, Wbi, n, blk):
    def body(p_r, hs_r, d_r, b0_r, wd_r, bd_r, a_r, wg_r, wbi_r,
             h1_r, hsg_r, tb_r):
        invn = _inv_norm(d_r)
        h1 = jax.nn.relu(invn * (p_r[0] + p_r[1] + hs_r[...]) + b0_r[...])
        h1_r[...] = h1
        hd = jnp.dot(h1, wd_r[...], preferred_element_type=jnp.float32) \
            + bd_r[...]
        hd = jnp.where(hd >= 0, hd, a_r[0, 0] * hd)
        hsg_r[...] = jnp.dot(hd, wg_r[...],
                             preferred_element_type=jnp.float32) * invn
        tb_r[...] = jnp.dot(hd, wbi_r[...], preferred_element_type=jnp.float32)

    return _row_call(
        body, n, blk,
        [(_sd((n, D)), 'r'), (_sd((n, D)), 'r'), (_sd((n, D)), 'r')],
        [(P, 'p'), (hs0, 'r'), (degp, 'd'), (b0, 'f'), (Wd, 'f'),
         (bd, 'f'), (a, 'f'), (Wg, 'f'), (Wbi, 'f')])


def tc_a2(h2, degp, Wd, bd, a, Wg, Wbi, n, blk):
    def body(h_r, d_r, wd_r, bd_r, a_r, wg_r, wbi_r, hsg_r, tb_r):
        invn = _inv_norm(d_r)
        hd = jnp.dot(h_r[...], wd_r[...],
                     preferred_element_type=jnp.float32) + bd_r[...]
        hd = jnp.where(hd >= 0, hd, a_r[0, 0] * hd)
        hsg_r[...] = jnp.dot(hd, wg_r[...],
                             preferred_element_type=jnp.float32) * invn
        tb_r[...] = jnp.dot(hd, wbi_r[...], preferred_element_type=jnp.float32)

    return _row_call(
        body, n, blk,
        [(_sd((n, D)), 'r'), (_sd((n, D)), 'r')],
        [(h2, 'r'), (degp, 'd'), (Wd, 'f'), (bd, 'f'), (a, 'f'),
         (Wg, 'f'), (Wbi, 'f')])


def tc_score(Q, hsg, tb, degp, bg, bbi, n, blk):
    def body(q_r, hsg_r, tb_r, d_r, bg_r, bbi_r, o_r):
        invn = _inv_norm(d_r)
        hx = invn * (q_r[0] + q_r[1] + hsg_r[...]) + bg_r[...]
        s = jnp.sum(tb_r[...] * hx, axis=1, keepdims=True) + bbi_r[0, 0]
        o_r[...] = jax.nn.sigmoid(s)

    return _row_call(
        body, n, blk, [(_sd((n, 1)), 'r')],
        [(Q, 'p'), (hsg, 'r'), (tb, 'r'), (degp, 'd'), (bg, 'f'),
         (bbi, 'f')])[0]


def tc_epilogue_mm(P, hs, degp, b, W, n, blk):
    def body(p_r, hs_r, d_r, b_r, w_r, o_r):
        invn = _inv_norm(d_r)
        hm = jax.nn.relu(invn * (p_r[0] + p_r[1] + hs_r[...]) + b_r[...])
        o_r[...] = jnp.dot(hm, w_r[...], preferred_element_type=jnp.float32)

    return _row_call(body, n, blk, [(_sd((n, D)), 'r')],
                     [(P, 'p'), (hs, 'r'), (degp, 'd'), (b, 'f'), (W, 'f')])[0]


def tc_gs_prep(mapping, degp, n, blk):
    def body(m_r, d_r, g_r, s_r):
        invn = _inv_norm(d_r)
        m = m_r[...]
        rows = (lax.broadcasted_iota(jnp.int32, (blk, 1), 0)
                + pl.program_id(0) * blk) & 4095
        g_r[...] = jnp.where(m >= 0, m, rows)
        s_r[...] = jnp.where(m >= 0, invn, 0.0)

    return _row_call(
        body, n, blk,
        [(_sd((n, 1), jnp.int32), 'r'), (_sd((n, 1)), 'r')],
        [(mapping, 'r'), (degp, 'd')])


def tc_scale(raw, scl, n, blk):
    def body(x_r, s_r, o_r):
        o_r[...] = x_r[...] * s_r[...]

    return _row_call(body, n, blk, [(_sd((n, D)), 'r')],
                     [(raw, 'r'), (scl, 'r')])[0]


def tc_cross(Rm, hsm, U, hsf, degp, bm, bu, W, n, blk):
    def body(rm_r, hsm_r, u_r, hsf_r, d_r, bm_r, bu_r, w_r, o_r):
        invn = _inv_norm(d_r)
        hm = jax.nn.relu(invn * (rm_r[0] + rm_r[1] + hsm_r[...]) + bm_r[...])
        hc = hm + invn * (u_r[0] + u_r[1] + hsf_r[...]) + bu_r[...]
        o_r[...] = jnp.dot(hc, w_r[...], preferred_element_type=jnp.float32)

    return _row_call(
        body, n, blk, [(_sd((n, D)), 'r')],
        [(Rm, 'p'), (hsm, 'r'), (U, 'p'), (hsf, 'r'), (degp, 'd'),
         (bm, 'f'), (bu, 'f'), (W, 'f')])[0]


def tc_cross_final(Rm, hsm, U, hsf, degp, bm, bu, h1, WeA, WeB, n, blk):
    def body(rm_r, hsm_r, u_r, hsf_r, d_r, bm_r, bu_r, h1_r, wa_r, wb_r, o_r):
        invn = _inv_norm(d_r)
        hm = jax.nn.relu(invn * (rm_r[0] + rm_r[1] + hsm_r[...]) + bm_r[...])
        hc = hm + invn * (u_r[0] + u_r[1] + hsf_r[...]) + bu_r[...]
        o_r[...] = (jnp.dot(h1_r[...], wa_r[...],
                            preferred_element_type=jnp.float32)
                    + jnp.dot(hc, wb_r[...],
                              preferred_element_type=jnp.float32)) * invn

    return _row_call(
        body, n, blk, [(_sd((n, D)), 'r')],
        [(Rm, 'p'), (hsm, 'r'), (U, 'p'), (hsf, 'r'), (degp, 'd'),
         (bm, 'f'), (bu, 'f'), (h1, 'r'), (WeA, 'f'), (WeB, 'f')])[0]


def tc_final(P, hs, degp, b, n, blk):
    def body(p_r, hs_r, d_r, b_r, o_r):
        invn = _inv_norm(d_r)
        o_r[...] = invn * (p_r[0] + p_r[1] + hs_r[...]) + b_r[...]

    return _row_call(body, n, blk, [(_sd((n, D)), 'r')],
                     [(P, 'p'), (hs, 'r'), (degp, 'd'), (b, 'f')])[0]


def kernel(x, params, edge_index):
    p = params
    src = edge_index[0]
    dst = edge_index[1]
    pad = EP - E
    srcc = jnp.concatenate(
        [src, jnp.arange(pad, dtype=jnp.int32) % 4096]).reshape(NW, CPT, ECH)
    dstc = jnp.concatenate(
        [dst, N + (jnp.arange(pad, dtype=jnp.int32) % 32)]).reshape(
        NW, CPT, ECH)

    gs = lambda v: v.reshape(v.shape[0] // GCH, 1, GCH)
    r2 = lambda v: v.reshape(v.shape[0], 1)
    rd = lambda dg: dg.reshape(2, dg.shape[1] * dg.shape[3], 1)
    b = {k_: p[k_].reshape(1, D) for k_ in
         ('b0', 'bd1', 'bg1', 'bd2', 'bg2', 'bm1', 'bm2', 'bm3',
          'bu1', 'bu2', 'be')}
    s = {k_: p[k_].reshape(1, 1) for k_ in ('a1', 'bbi1', 'a2', 'bbi2')}

    degp1 = rd(sc_deg(dstc, NPAD1))

    hs0 = tc_mm_scale(x, p['W0'], degp1, N, 1000)
    P0 = sc_edge_agg(hs0, srcc, dstc, N, NPAD1)
    h1, hsg1, tb1 = tc_a1(P0, hs0, degp1, b['b0'], p['Wd1'], b['bd1'],
                          s['a1'], p['Wg1'], p['Wbi1'], N, 1000)
    Q1 = sc_edge_agg(hsg1, srcc, dstc, N, NPAD1)
    probs1 = tc_score(Q1, hsg1, tb1, degp1, b['bg1'], s['bbi1'], N, 1000)
    idx1, sc1, map1 = sc_topk(probs1.reshape(N), N, K1)

    srcc2, dstc2, degp2 = sc_relabel(map1, srcc, dstc, N, K1, NPAD2, 63)
    degp2 = rd(degp2)
    h2 = tc_scale(sc_gather(h1, gs(idx1), K1).reshape(K1, D),
                  r2(sc1), K1, 1000)

    hsg2, tb2 = tc_a2(h2, degp2, p['Wd2'], b['bd2'], s['a2'], p['Wg2'],
                      p['Wbi2'], K1, 1000)
    Q2 = sc_edge_agg(hsg2, srcc2, dstc2, K1, NPAD2)
    probs2 = tc_score(Q2, hsg2, tb2, degp2, b['bg2'], s['bbi2'], K1, 1000)
    idx2, sc2, map2 = sc_topk(probs2.reshape(K1), K1, K2)

    srcc3, dstc3, degp3 = sc_relabel(map2, srcc2, dstc2, K1, K2, NPAD3, 127)
    degp3 = rd(degp3)
    h3 = tc_scale(sc_gather(h2, gs(idx2), K2).reshape(K2, D),
                  r2(sc2), K2, 800)

    hsm1 = tc_mm_scale(h1, p['Wm1'], degp1, N, 1000)
    R1 = sc_edge_agg(hsm1, srcc, dstc, N, NPAD1)
    hsm2 = tc_mm_scale(h2, p['Wm2'], degp2, K1, 1000)
    R2 = sc_edge_agg(hsm2, srcc2, dstc2, K1, NPAD2)
    hsm3 = tc_mm_scale(h3, p['Wm3'], degp3, K2, 800)
    R3 = sc_edge_agg(hsm3, srcc3, dstc3, K2, NPAD3)

    fw3 = tc_epilogue_mm(R3, hsm3, degp3, b['bm3'], p['Wu2'], K2, 800)
    gidx2, scl2 = tc_gs_prep(r2(map2[:K1]), degp2, K1, 1000)
    hsf2 = tc_scale(sc_gather(fw3, gs(gidx2.reshape(K1)), K1).reshape(K1, D),
                    scl2, K1, 1000)
    U2 = sc_edge_agg(hsf2, srcc2, dstc2, K1, NPAD2)

    fw2 = tc_cross(R2, hsm2, U2, hsf2, degp2, b['bm2'], b['bu2'],
                   p['Wu1'], K1, 1000)
    gidx1, scl1 = tc_gs_prep(r2(map1[:N]), degp1, N, 1000)
    hsf1 = tc_scale(sc_gather(fw2, gs(gidx1.reshape(N)), N).reshape(N, D),
                    scl1, N, 1000)
    U1 = sc_edge_agg(hsf1, srcc, dstc, N, NPAD1)

    hse = tc_cross_final(R1, hsm1, U1, hsf1, degp1, b['bm1'], b['bu1'],
                         h1, p['We'][:D], p['We'][D:], N, 1000)
    Pe = sc_edge_agg(hse, srcc, dstc, N, NPAD1)
    out = tc_final(Pe, hse, degp1, b['be'], N, 1000)
    return out
